# Initial kernel scaffold; baseline (speedup 1.0000x reference)
import jax
import jax.numpy as jnp
from jax.experimental import pallas as pl


def kernel(x, edge_index, mask, W1, b1, W2, b2):
    raise NotImplementedError("write your pallas kernel here")


# speedup vs baseline: 7.5604x; 7.5604x over previous
import functools

import jax
import jax.numpy as jnp
from jax import lax
from jax.experimental import pallas as pl
from jax.experimental.pallas import tpu as pltpu
from jax.experimental.pallas import tpu_sc as plsc

N = 10000
E = 160000
D = 256
H = 256
C = 64

NSUB = 16
EPAD = 163840
EROWS = EPAD // 128
ROWS_PER_SUB = EROWS // NSUB
CHUNK_ROWS = 8
NCHUNK = ROWS_PER_SUB // CHUNK_ROWS
NR = 10112
WB = 624
WBTAIL = N - WB * NSUB
DEGWB = NR // NSUB

_HI = jax.lax.Precision.HIGHEST

@functools.lru_cache(maxsize=None)
def _vector_mesh():
  return plsc.VectorSubcoreMesh(
      core_axis_name="c", subcore_axis_name="s", num_cores=2, num_subcores=NSUB)


def _sc_degree(dst2d, ones2d, zeros2d):
  @functools.partial(
      pl.kernel,
      out_type=jax.ShapeDtypeStruct((NR, 16), jnp.float32),
      mesh=_vector_mesh(),
      compiler_params=pltpu.CompilerParams(use_tc_tiling_on_sc=False),
      scratch_types=[
          pltpu.VMEM((CHUNK_ROWS, 128), jnp.int32),
          pltpu.VMEM((128, 16), jnp.float32),
          pltpu.VMEM_SHARED((NR, 16), jnp.float32),
      ],
  )
  def deg_kernel(dst_hbm, ones_hbm, zeros_hbm, out_hbm, dst_v, ones_v, acc_sh):
    c = lax.axis_index("c")
    s = lax.axis_index("s")

    @pl.when(c == 0)
    def _():
      pltpu.sync_copy(ones_hbm, ones_v)
      pltpu.sync_copy(zeros_hbm.at[pl.ds(s * DEGWB, DEGWB)],
                      acc_sh.at[pl.ds(s * DEGWB, DEGWB)])
      plsc.subcore_barrier()

      base = s * ROWS_PER_SUB

      @pl.loop(0, NCHUNK)
      def _(k):
        pltpu.sync_copy(dst_hbm.at[pl.ds(base + k * CHUNK_ROWS, CHUNK_ROWS)],
                        dst_v)

        @pl.loop(0, CHUNK_ROWS)
        def _(j):
          pltpu.sync_copy(ones_v, acc_sh.at[dst_v.at[j]], add=True)

      plsc.subcore_barrier()
      pltpu.sync_copy(acc_sh.at[pl.ds(s * DEGWB, DEGWB)],
                      out_hbm.at[pl.ds(s * DEGWB, DEGWB)])

  return deg_kernel(dst2d, ones2d, zeros2d)


def _make_sc_agg(fc):
  cp = (None if fc % 128 == 0
        else pltpu.CompilerParams(use_tc_tiling_on_sc=False))

  @functools.partial(
      pl.kernel,
      out_type=(jax.ShapeDtypeStruct((N, fc), jnp.float32),
                jax.ShapeDtypeStruct((N, fc), jnp.float32)),
      mesh=_vector_mesh(),
      compiler_params=cp,
      scratch_types=[
          pltpu.VMEM((CHUNK_ROWS, 128), jnp.int32),
          pltpu.VMEM((CHUNK_ROWS, 128), jnp.int32),
          pltpu.VMEM((128, fc), jnp.float32),
          pltpu.VMEM_SHARED((NR, fc), jnp.float32),
      ],
  )
  def agg_kernel(h0_hbm, h1_hbm, src_hbm, dst_hbm, out0_hbm, out1_hbm,
                 src_v, dst_v, rows_v, acc_sh):
    c = lax.axis_index("c")
    s = lax.axis_index("s")

    def run(h_hbm, out_hbm):
      pltpu.sync_copy(h_hbm.at[pl.ds(s * WB, WB)], acc_sh.at[pl.ds(s * WB, WB)])

      @pl.when(s == 0)
      def _():
        pltpu.sync_copy(h_hbm.at[pl.ds(WB * NSUB, WBTAIL)],
                        acc_sh.at[pl.ds(WB * NSUB, WBTAIL)])

      plsc.subcore_barrier()

      base = s * ROWS_PER_SUB

      @pl.loop(0, NCHUNK)
      def _(k):
        pltpu.sync_copy(src_hbm.at[pl.ds(base + k * CHUNK_ROWS, CHUNK_ROWS)],
                        src_v)
        pltpu.sync_copy(dst_hbm.at[pl.ds(base + k * CHUNK_ROWS, CHUNK_ROWS)],
                        dst_v)

        @pl.loop(0, CHUNK_ROWS)
        def _(j):
          pltpu.sync_copy(h_hbm.at[src_v.at[j]], rows_v)
          pltpu.sync_copy(rows_v, acc_sh.at[dst_v.at[j]], add=True)

      plsc.subcore_barrier()
      pltpu.sync_copy(acc_sh.at[pl.ds(s * WB, WB)], out_hbm.at[pl.ds(s * WB, WB)])

      @pl.when(s == 0)
      def _():
        pltpu.sync_copy(acc_sh.at[pl.ds(WB * NSUB, WBTAIL)],
                        out_hbm.at[pl.ds(WB * NSUB, WBTAIL)])

    @pl.when(c == 0)
    def _():
      run(h0_hbm, out0_hbm)

    @pl.when(c == 1)
    def _():
      run(h1_hbm, out1_hbm)

  return agg_kernel


_make_sc_agg = functools.lru_cache(maxsize=None)(_make_sc_agg)


_BR = 1000


def _mm1_body(x_ref, w_ref, o_ref):
  o = lax.dot_general(x_ref[...], w_ref[...], (((1,), (0,)), ((), ())),
                      precision=_HI, preferred_element_type=jnp.float32)
  o_ref[0] = o[:, :128]
  o_ref[1] = o[:, 128:]


def _tc_matmul1(x, w1):
  return pl.pallas_call(
      _mm1_body,
      grid=(N // _BR,),
      in_specs=[
          pl.BlockSpec((_BR, D), lambda i: (i, 0)),
          pl.BlockSpec((D, H), lambda i: (0, 0)),
      ],
      out_specs=pl.BlockSpec((2, _BR, 128), lambda i: (0, i, 0)),
      out_shape=jax.ShapeDtypeStruct((2, N, 128), jnp.float32),
  )(x, w1)


def _scale_body(xw_ref, deg_ref, o0_ref, o1_ref):
  dinv = lax.rsqrt(deg_ref[:, 0:1] + 1.0)
  o0_ref[...] = xw_ref[0] * dinv
  o1_ref[...] = xw_ref[1] * dinv


def _tc_scale(xw, deg):
  return pl.pallas_call(
      _scale_body,
      grid=(N // _BR,),
      in_specs=[
          pl.BlockSpec((2, _BR, 128), lambda i: (0, i, 0)),
          pl.BlockSpec((_BR, 16), lambda i: (i, 0)),
      ],
      out_specs=[
          pl.BlockSpec((_BR, 128), lambda i: (i, 0)),
          pl.BlockSpec((_BR, 128), lambda i: (i, 0)),
      ],
      out_shape=[jax.ShapeDtypeStruct((N, 128), jnp.float32),
                 jax.ShapeDtypeStruct((N, 128), jnp.float32)],
  )(xw, deg)


def _layer2_body(a0_ref, a1_ref, deg_ref, b1_ref, w2_ref, o0_ref, o1_ref):
  dinv = lax.rsqrt(deg_ref[:, 0:1] + 1.0)
  h0 = jax.nn.relu(a0_ref[...] * dinv + b1_ref[0:1, :128])
  h1 = jax.nn.relu(a1_ref[...] * dinv + b1_ref[0:1, 128:])
  o = lax.dot_general(h0, w2_ref[:128, :], (((1,), (0,)), ((), ())),
                      precision=_HI, preferred_element_type=jnp.float32)
  o += lax.dot_general(h1, w2_ref[128:, :], (((1,), (0,)), ((), ())),
                       precision=_HI, preferred_element_type=jnp.float32)
  o = o * dinv
  o0_ref[...] = o[:, :32]
  o1_ref[...] = o[:, 32:]


def _tc_layer2(a0, a1, deg, b1, w2):
  return pl.pallas_call(
      _layer2_body,
      grid=(N // _BR,),
      in_specs=[
          pl.BlockSpec((_BR, 128), lambda i: (i, 0)),
          pl.BlockSpec((_BR, 128), lambda i: (i, 0)),
          pl.BlockSpec((_BR, 16), lambda i: (i, 0)),
          pl.BlockSpec((1, H), lambda i: (0, 0)),
          pl.BlockSpec((H, C), lambda i: (0, 0)),
      ],
      out_specs=[
          pl.BlockSpec((_BR, 32), lambda i: (i, 0)),
          pl.BlockSpec((_BR, 32), lambda i: (i, 0)),
      ],
      out_shape=[jax.ShapeDtypeStruct((N, 32), jnp.float32),
                 jax.ShapeDtypeStruct((N, 32), jnp.float32)],
  )(a0, a1, deg, b1, w2)


def _finalize_body(a0_ref, a1_ref, deg_ref, b2_ref, logp_ref, outn_ref):
  dinv = lax.rsqrt(deg_ref[:, 0:1] + 1.0)
  o = jnp.concatenate([a0_ref[...], a1_ref[...]], axis=1) * dinv + b2_ref[0:1, :]
  m = jnp.max(o, axis=1, keepdims=True)
  sh = o - m
  lse = jnp.log(jnp.sum(jnp.exp(sh), axis=1, keepdims=True))
  logp_ref[...] = sh - lse
  sq = jnp.sum(o * o, axis=1, keepdims=True)
  outn_ref[...] = o * lax.rsqrt(sq)


def _tc_finalize(a0, a1, deg, b2):
  return pl.pallas_call(
      _finalize_body,
      grid=(N // _BR,),
      in_specs=[
          pl.BlockSpec((_BR, 32), lambda i: (i, 0)),
          pl.BlockSpec((_BR, 32), lambda i: (i, 0)),
          pl.BlockSpec((_BR, 16), lambda i: (i, 0)),
          pl.BlockSpec((1, C), lambda i: (0, 0)),
      ],
      out_specs=[
          pl.BlockSpec((_BR, C), lambda i: (i, 0)),
          pl.BlockSpec((_BR, C), lambda i: (i, 0)),
      ],
      out_shape=[jax.ShapeDtypeStruct((N, C), jnp.float32),
                 jax.ShapeDtypeStruct((N, C), jnp.float32)],
  )(a0, a1, deg, b2)


_BS = 200


def _sim_body(lhs_ref, rhst_ref, mask_ref, o_ref):
  s = lax.dot_general(lhs_ref[...], rhst_ref[...], (((1,), (0,)), ((), ())),
                      precision=_HI, preferred_element_type=jnp.float32)
  o_ref[...] = (1.0 - mask_ref[...]) * s


def _tc_sim(outn, outn_t, mask):
  return pl.pallas_call(
      _sim_body,
      grid=(N // _BS,),
      in_specs=[
          pl.BlockSpec((_BS, C), lambda i: (i, 0)),
          pl.BlockSpec((C, N), lambda i: (0, 0)),
          pl.BlockSpec((_BS, N), lambda i: (i, 0)),
      ],
      out_specs=pl.BlockSpec((_BS, N), lambda i: (i, 0)),
      out_shape=jax.ShapeDtypeStruct((N, N), jnp.float32),
  )(outn, outn_t, mask)


def kernel(x, edge_index, mask, W1, b1, W2, b2):
  src = edge_index[0].astype(jnp.int32)
  dst = edge_index[1].astype(jnp.int32)

  extra = EPAD - E
  pad_ids = jnp.arange(extra, dtype=jnp.int32)
  src_p = jnp.concatenate([src, pad_ids % N]).reshape(EROWS, 128)
  dst_p = jnp.concatenate([dst, N + pad_ids % (NR - N)]).reshape(EROWS, 128)

  ones2d = jnp.ones((128, 16), jnp.float32)
  zeros2d = jnp.zeros((NR, 16), jnp.float32)
  deg16 = _sc_degree(dst_p, ones2d, zeros2d)[:N]

  xw = _tc_matmul1(x, W1)
  hs1_0, hs1_1 = _tc_scale(xw, deg16)
  a1_0, a1_1 = _make_sc_agg(128)(hs1_0, hs1_1, src_p, dst_p)
  hs2_0, hs2_1 = _tc_layer2(a1_0, a1_1, deg16, b1.reshape(1, H), W2)
  a2_0, a2_1 = _make_sc_agg(32)(hs2_0, hs2_1, src_p, dst_p)
  logp, outn = _tc_finalize(a2_0, a2_1, deg16, b2.reshape(1, C))
  x_dis = _tc_sim(outn, outn.T, mask)
  return (logp, x_dis)


# speedup vs baseline: 9.0308x; 1.1945x over previous
import functools

import jax
import jax.numpy as jnp
from jax import lax
from jax.experimental import pallas as pl
from jax.experimental.pallas import tpu as pltpu
from jax.experimental.pallas import tpu_sc as plsc

N = 10000
E = 160000
D = 256
H = 256
C = 64

NSUB = 16
EPAD = 163840
EROWS = EPAD // 128
ROWS_PER_SUB = EROWS // NSUB
CHUNK_ROWS = 8
NCHUNK = ROWS_PER_SUB // CHUNK_ROWS
NRD = 10112
NRA = 10016
IDXROWS = ROWS_PER_SUB // 2
WB = 624
WBTAIL = N - WB * NSUB
DEGWB = NRD // NSUB

_HI = jax.lax.Precision.HIGHEST

@functools.lru_cache(maxsize=None)
def _vector_mesh():
  return plsc.VectorSubcoreMesh(
      core_axis_name="c", subcore_axis_name="s", num_cores=2, num_subcores=NSUB)


def _sc_degree(dst2d, ones2d, zeros2d):
  @functools.partial(
      pl.kernel,
      out_type=jax.ShapeDtypeStruct((NRD, 16), jnp.float32),
      mesh=_vector_mesh(),
      compiler_params=pltpu.CompilerParams(use_tc_tiling_on_sc=False),
      scratch_types=[
          pltpu.VMEM((CHUNK_ROWS, 128), jnp.int32),
          pltpu.VMEM((128, 16), jnp.float32),
          pltpu.VMEM_SHARED((NRD, 16), jnp.float32),
      ],
  )
  def deg_kernel(dst_hbm, ones_hbm, zeros_hbm, out_hbm, dst_v, ones_v, acc_sh):
    c = lax.axis_index("c")
    s = lax.axis_index("s")

    @pl.when(c == 0)
    def _():
      pltpu.sync_copy(ones_hbm, ones_v)
      pltpu.sync_copy(zeros_hbm.at[pl.ds(s * DEGWB, DEGWB)],
                      acc_sh.at[pl.ds(s * DEGWB, DEGWB)])
      plsc.subcore_barrier()

      base = s * ROWS_PER_SUB

      @pl.loop(0, NCHUNK)
      def _(k):
        pltpu.sync_copy(dst_hbm.at[pl.ds(base + k * CHUNK_ROWS, CHUNK_ROWS)],
                        dst_v)

        @pl.loop(0, CHUNK_ROWS)
        def _(j):
          pltpu.sync_copy(ones_v, acc_sh.at[dst_v.at[j]], add=True)

      plsc.subcore_barrier()
      pltpu.sync_copy(acc_sh.at[pl.ds(s * DEGWB, DEGWB)],
                      out_hbm.at[pl.ds(s * DEGWB, DEGWB)])

  return deg_kernel(dst2d, ones2d, zeros2d)


def _make_sc_agg(fc):
  cp = (None if fc % 128 == 0
        else pltpu.CompilerParams(use_tc_tiling_on_sc=False))

  @functools.partial(
      pl.kernel,
      out_type=(jax.ShapeDtypeStruct((N, fc), jnp.float32),
                jax.ShapeDtypeStruct((N, fc), jnp.float32)),
      mesh=_vector_mesh(),
      compiler_params=cp,
      scratch_types=[
          pltpu.VMEM((IDXROWS, 128), jnp.int32),
          pltpu.VMEM((IDXROWS, 128), jnp.int32),
          pltpu.VMEM((128, fc), jnp.float32),
          pltpu.VMEM((128, fc), jnp.float32),
          pltpu.VMEM_SHARED((NRA, fc), jnp.float32),
          pltpu.SemaphoreType.DMA,
          pltpu.SemaphoreType.DMA,
      ],
  )
  def agg_kernel(h0_hbm, h1_hbm, src_hbm, dst_hbm, out0_hbm, out1_hbm,
                 src_v, dst_v, rows_a, rows_b, acc_sh, sem_a, sem_b):
    c = lax.axis_index("c")
    s = lax.axis_index("s")

    def run(h_hbm, out_hbm):
      pltpu.sync_copy(h_hbm.at[pl.ds(s * WB, WB)], acc_sh.at[pl.ds(s * WB, WB)])

      @pl.when(s == 0)
      def _():
        pltpu.sync_copy(h_hbm.at[pl.ds(WB * NSUB, WBTAIL)],
                        acc_sh.at[pl.ds(WB * NSUB, WBTAIL)])

      plsc.subcore_barrier()

      def start(r, buf, sem):
        pltpu.async_copy(h_hbm.at[src_v.at[r]], buf, sem)

      def wait(buf, sem):
        pltpu.make_async_copy(h_hbm.at[pl.ds(0, 128)], buf, sem).wait()

      def scat(r, buf):
        pltpu.sync_copy(buf, acc_sh.at[dst_v.at[r]], add=True)

      @pl.loop(0, 2)
      def _(p):
        base = s * ROWS_PER_SUB + p * IDXROWS
        pltpu.sync_copy(src_hbm.at[pl.ds(base, IDXROWS)], src_v)
        pltpu.sync_copy(dst_hbm.at[pl.ds(base, IDXROWS)], dst_v)
        start(0, rows_a, sem_a)

        @pl.loop(0, IDXROWS, step=2)
        def _(r):
          start(r + 1, rows_b, sem_b)
          wait(rows_a, sem_a)
          scat(r, rows_a)

          @pl.when(r + 2 < IDXROWS)
          def _():
            start(r + 2, rows_a, sem_a)

          wait(rows_b, sem_b)
          scat(r + 1, rows_b)

      plsc.subcore_barrier()
      pltpu.sync_copy(acc_sh.at[pl.ds(s * WB, WB)], out_hbm.at[pl.ds(s * WB, WB)])

      @pl.when(s == 0)
      def _():
        pltpu.sync_copy(acc_sh.at[pl.ds(WB * NSUB, WBTAIL)],
                        out_hbm.at[pl.ds(WB * NSUB, WBTAIL)])

    @pl.when(c == 0)
    def _():
      run(h0_hbm, out0_hbm)

    @pl.when(c == 1)
    def _():
      run(h1_hbm, out1_hbm)

  return agg_kernel


_make_sc_agg = functools.lru_cache(maxsize=None)(_make_sc_agg)


_BR = 1000


def _mm1_body(x_ref, w_ref, o_ref):
  o = lax.dot_general(x_ref[...], w_ref[...], (((1,), (0,)), ((), ())),
                      precision=_HI, preferred_element_type=jnp.float32)
  o_ref[0] = o[:, :128]
  o_ref[1] = o[:, 128:]


def _tc_matmul1(x, w1):
  return pl.pallas_call(
      _mm1_body,
      grid=(N // _BR,),
      in_specs=[
          pl.BlockSpec((_BR, D), lambda i: (i, 0)),
          pl.BlockSpec((D, H), lambda i: (0, 0)),
      ],
      out_specs=pl.BlockSpec((2, _BR, 128), lambda i: (0, i, 0)),
      out_shape=jax.ShapeDtypeStruct((2, N, 128), jnp.float32),
  )(x, w1)


def _scale_body(xw_ref, deg_ref, o0_ref, o1_ref):
  dinv = lax.rsqrt(deg_ref[:, 0:1] + 1.0)
  o0_ref[...] = xw_ref[0] * dinv
  o1_ref[...] = xw_ref[1] * dinv


def _tc_scale(xw, deg):
  return pl.pallas_call(
      _scale_body,
      grid=(N // _BR,),
      in_specs=[
          pl.BlockSpec((2, _BR, 128), lambda i: (0, i, 0)),
          pl.BlockSpec((_BR, 16), lambda i: (i, 0)),
      ],
      out_specs=[
          pl.BlockSpec((_BR, 128), lambda i: (i, 0)),
          pl.BlockSpec((_BR, 128), lambda i: (i, 0)),
      ],
      out_shape=[jax.ShapeDtypeStruct((N, 128), jnp.float32),
                 jax.ShapeDtypeStruct((N, 128), jnp.float32)],
  )(xw, deg)


def _layer2_body(a0_ref, a1_ref, deg_ref, b1_ref, w2_ref, o0_ref, o1_ref):
  dinv = lax.rsqrt(deg_ref[:, 0:1] + 1.0)
  h0 = jax.nn.relu(a0_ref[...] * dinv + b1_ref[0:1, :128])
  h1 = jax.nn.relu(a1_ref[...] * dinv + b1_ref[0:1, 128:])
  o = lax.dot_general(h0, w2_ref[:128, :], (((1,), (0,)), ((), ())),
                      precision=_HI, preferred_element_type=jnp.float32)
  o += lax.dot_general(h1, w2_ref[128:, :], (((1,), (0,)), ((), ())),
                       precision=_HI, preferred_element_type=jnp.float32)
  o = o * dinv
  o0_ref[...] = o[:, :32]
  o1_ref[...] = o[:, 32:]


def _tc_layer2(a0, a1, deg, b1, w2):
  return pl.pallas_call(
      _layer2_body,
      grid=(N // _BR,),
      in_specs=[
          pl.BlockSpec((_BR, 128), lambda i: (i, 0)),
          pl.BlockSpec((_BR, 128), lambda i: (i, 0)),
          pl.BlockSpec((_BR, 16), lambda i: (i, 0)),
          pl.BlockSpec((1, H), lambda i: (0, 0)),
          pl.BlockSpec((H, C), lambda i: (0, 0)),
      ],
      out_specs=[
          pl.BlockSpec((_BR, 32), lambda i: (i, 0)),
          pl.BlockSpec((_BR, 32), lambda i: (i, 0)),
      ],
      out_shape=[jax.ShapeDtypeStruct((N, 32), jnp.float32),
                 jax.ShapeDtypeStruct((N, 32), jnp.float32)],
  )(a0, a1, deg, b1, w2)


def _finalize_body(a0_ref, a1_ref, deg_ref, b2_ref, logp_ref, outn_ref):
  dinv = lax.rsqrt(deg_ref[:, 0:1] + 1.0)
  o = jnp.concatenate([a0_ref[...], a1_ref[...]], axis=1) * dinv + b2_ref[0:1, :]
  m = jnp.max(o, axis=1, keepdims=True)
  sh = o - m
  lse = jnp.log(jnp.sum(jnp.exp(sh), axis=1, keepdims=True))
  logp_ref[...] = sh - lse
  sq = jnp.sum(o * o, axis=1, keepdims=True)
  outn_ref[...] = o * lax.rsqrt(sq)


def _tc_finalize(a0, a1, deg, b2):
  return pl.pallas_call(
      _finalize_body,
      grid=(N // _BR,),
      in_specs=[
          pl.BlockSpec((_BR, 32), lambda i: (i, 0)),
          pl.BlockSpec((_BR, 32), lambda i: (i, 0)),
          pl.BlockSpec((_BR, 16), lambda i: (i, 0)),
          pl.BlockSpec((1, C), lambda i: (0, 0)),
      ],
      out_specs=[
          pl.BlockSpec((_BR, C), lambda i: (i, 0)),
          pl.BlockSpec((_BR, C), lambda i: (i, 0)),
      ],
      out_shape=[jax.ShapeDtypeStruct((N, C), jnp.float32),
                 jax.ShapeDtypeStruct((N, C), jnp.float32)],
  )(a0, a1, deg, b2)


_BS = 200


def _sim_body(lhs_ref, rhst_ref, mask_ref, o_ref):
  s = lax.dot_general(lhs_ref[...], rhst_ref[...], (((1,), (0,)), ((), ())),
                      precision=_HI, preferred_element_type=jnp.float32)
  o_ref[...] = (1.0 - mask_ref[...]) * s


def _tc_sim(outn, outn_t, mask):
  return pl.pallas_call(
      _sim_body,
      grid=(N // _BS,),
      in_specs=[
          pl.BlockSpec((_BS, C), lambda i: (i, 0)),
          pl.BlockSpec((C, N), lambda i: (0, 0)),
          pl.BlockSpec((_BS, N), lambda i: (i, 0)),
      ],
      out_specs=pl.BlockSpec((_BS, N), lambda i: (i, 0)),
      out_shape=jax.ShapeDtypeStruct((N, N), jnp.float32),
  )(outn, outn_t, mask)


def kernel(x, edge_index, mask, W1, b1, W2, b2):
  src = edge_index[0].astype(jnp.int32)
  dst = edge_index[1].astype(jnp.int32)

  extra = EPAD - E
  pad_ids = jnp.arange(extra, dtype=jnp.int32)
  src_p = jnp.concatenate([src, pad_ids % N]).reshape(EROWS, 128)
  dst_p = jnp.concatenate([dst, N + pad_ids % (NRA - N)]).reshape(EROWS, 128)

  ones2d = jnp.ones((128, 16), jnp.float32)
  zeros2d = jnp.zeros((NRD, 16), jnp.float32)
  deg16 = _sc_degree(dst_p, ones2d, zeros2d)[:N]

  xw = _tc_matmul1(x, W1)
  hs1_0, hs1_1 = _tc_scale(xw, deg16)
  a1_0, a1_1 = _make_sc_agg(128)(hs1_0, hs1_1, src_p, dst_p)
  hs2_0, hs2_1 = _tc_layer2(a1_0, a1_1, deg16, b1.reshape(1, H), W2)
  a2_0, a2_1 = _make_sc_agg(32)(hs2_0, hs2_1, src_p, dst_p)
  logp, outn = _tc_finalize(a2_0, a2_1, deg16, b2.reshape(1, C))
  x_dis = _tc_sim(outn, outn.T, mask)
  return (logp, x_dis)


# speedup vs baseline: 10.2286x; 1.1326x over previous
import functools

import jax
import jax.numpy as jnp
from jax import lax
from jax.experimental import pallas as pl
from jax.experimental.pallas import tpu as pltpu
from jax.experimental.pallas import tpu_sc as plsc

N = 10000
E = 160000
D = 256
H = 256
C = 64

NSUB = 16
EPAD = 163840
EROWS = EPAD // 128
ROWS_PER_SUB = EROWS // NSUB
CHUNK_ROWS = 8
NCHUNK = ROWS_PER_SUB // CHUNK_ROWS
NRD = 10112
NRA = 10016
IDXROWS = ROWS_PER_SUB // 2
WB = 624
WBTAIL = N - WB * NSUB
DEGWB = NRD // NSUB

_HI = jax.lax.Precision.HIGHEST

@functools.lru_cache(maxsize=None)
def _vector_mesh():
  return plsc.VectorSubcoreMesh(
      core_axis_name="c", subcore_axis_name="s", num_cores=2, num_subcores=NSUB)


def _sc_degree(dst2d, ones2d, zeros2d):
  @functools.partial(
      pl.kernel,
      out_type=jax.ShapeDtypeStruct((NRD, 16), jnp.float32),
      mesh=_vector_mesh(),
      compiler_params=pltpu.CompilerParams(use_tc_tiling_on_sc=False),
      scratch_types=[
          pltpu.VMEM((CHUNK_ROWS, 128), jnp.int32),
          pltpu.VMEM((128, 16), jnp.float32),
          pltpu.VMEM_SHARED((NRD, 16), jnp.float32),
      ],
  )
  def deg_kernel(dst_hbm, ones_hbm, zeros_hbm, out_hbm, dst_v, ones_v, acc_sh):
    c = lax.axis_index("c")
    s = lax.axis_index("s")

    @pl.when(c == 0)
    def _():
      pltpu.sync_copy(ones_hbm, ones_v)
      pltpu.sync_copy(zeros_hbm.at[pl.ds(s * DEGWB, DEGWB)],
                      acc_sh.at[pl.ds(s * DEGWB, DEGWB)])
      plsc.subcore_barrier()

      base = s * ROWS_PER_SUB

      @pl.loop(0, NCHUNK)
      def _(k):
        pltpu.sync_copy(dst_hbm.at[pl.ds(base + k * CHUNK_ROWS, CHUNK_ROWS)],
                        dst_v)

        @pl.loop(0, CHUNK_ROWS)
        def _(j):
          pltpu.sync_copy(ones_v, acc_sh.at[dst_v.at[j]], add=True)

      plsc.subcore_barrier()
      pltpu.sync_copy(acc_sh.at[pl.ds(s * DEGWB, DEGWB)],
                      out_hbm.at[pl.ds(s * DEGWB, DEGWB)])

  return deg_kernel(dst2d, ones2d, zeros2d)


def _make_sc_agg(fc):
  cp = (None if fc % 128 == 0
        else pltpu.CompilerParams(use_tc_tiling_on_sc=False))

  @functools.partial(
      pl.kernel,
      out_type=(jax.ShapeDtypeStruct((N, fc), jnp.float32),
                jax.ShapeDtypeStruct((N, fc), jnp.float32)),
      mesh=_vector_mesh(),
      compiler_params=cp,
      scratch_types=[
          pltpu.VMEM((IDXROWS, 128), jnp.int32),
          pltpu.VMEM((IDXROWS, 128), jnp.int32),
          pltpu.VMEM((128, fc), jnp.float32),
          pltpu.VMEM((128, fc), jnp.float32),
          pltpu.VMEM_SHARED((NRA, fc), jnp.float32),
          pltpu.SemaphoreType.DMA,
          pltpu.SemaphoreType.DMA,
      ],
  )
  def agg_kernel(h0_hbm, h1_hbm, src_hbm, dst_hbm, out0_hbm, out1_hbm,
                 src_v, dst_v, rows_a, rows_b, acc_sh, sem_a, sem_b):
    c = lax.axis_index("c")
    s = lax.axis_index("s")

    def run(h_hbm, out_hbm):
      pltpu.sync_copy(h_hbm.at[pl.ds(s * WB, WB)], acc_sh.at[pl.ds(s * WB, WB)])

      @pl.when(s == 0)
      def _():
        pltpu.sync_copy(h_hbm.at[pl.ds(WB * NSUB, WBTAIL)],
                        acc_sh.at[pl.ds(WB * NSUB, WBTAIL)])

      plsc.subcore_barrier()

      def start(r, buf, sem):
        pltpu.async_copy(h_hbm.at[src_v.at[r]], buf, sem)

      def wait(buf, sem):
        pltpu.make_async_copy(h_hbm.at[pl.ds(0, 128)], buf, sem).wait()

      def scat(r, buf):
        pltpu.sync_copy(buf, acc_sh.at[dst_v.at[r]], add=True)

      @pl.loop(0, 2)
      def _(p):
        base = s * ROWS_PER_SUB + p * IDXROWS
        pltpu.sync_copy(src_hbm.at[pl.ds(base, IDXROWS)], src_v)
        pltpu.sync_copy(dst_hbm.at[pl.ds(base, IDXROWS)], dst_v)
        start(0, rows_a, sem_a)

        @pl.loop(0, IDXROWS, step=2)
        def _(r):
          start(r + 1, rows_b, sem_b)
          wait(rows_a, sem_a)
          scat(r, rows_a)

          @pl.when(r + 2 < IDXROWS)
          def _():
            start(r + 2, rows_a, sem_a)

          wait(rows_b, sem_b)
          scat(r + 1, rows_b)

      plsc.subcore_barrier()
      pltpu.sync_copy(acc_sh.at[pl.ds(s * WB, WB)], out_hbm.at[pl.ds(s * WB, WB)])

      @pl.when(s == 0)
      def _():
        pltpu.sync_copy(acc_sh.at[pl.ds(WB * NSUB, WBTAIL)],
                        out_hbm.at[pl.ds(WB * NSUB, WBTAIL)])

    @pl.when(c == 0)
    def _():
      run(h0_hbm, out0_hbm)

    @pl.when(c == 1)
    def _():
      run(h1_hbm, out1_hbm)

  return agg_kernel


_make_sc_agg = functools.lru_cache(maxsize=None)(_make_sc_agg)


_BR = 1000


def _mm1_body(x_ref, w_ref, o_ref):
  o = lax.dot_general(x_ref[...], w_ref[...], (((1,), (0,)), ((), ())),
                      precision=_HI, preferred_element_type=jnp.float32)
  o_ref[0] = o[:, :128]
  o_ref[1] = o[:, 128:]


def _tc_matmul1(x, w1):
  return pl.pallas_call(
      _mm1_body,
      grid=(N // _BR,),
      in_specs=[
          pl.BlockSpec((_BR, D), lambda i: (i, 0)),
          pl.BlockSpec((D, H), lambda i: (0, 0)),
      ],
      out_specs=pl.BlockSpec((2, _BR, 128), lambda i: (0, i, 0)),
      out_shape=jax.ShapeDtypeStruct((2, N, 128), jnp.float32),
  )(x, w1)


def _scale_body(xw_ref, deg_ref, o0_ref, o1_ref):
  dinv = lax.rsqrt(deg_ref[:, 0:1] + 1.0)
  o0_ref[...] = xw_ref[0] * dinv
  o1_ref[...] = xw_ref[1] * dinv


def _tc_scale(xw, deg):
  return pl.pallas_call(
      _scale_body,
      grid=(N // _BR,),
      in_specs=[
          pl.BlockSpec((2, _BR, 128), lambda i: (0, i, 0)),
          pl.BlockSpec((_BR, 16), lambda i: (i, 0)),
      ],
      out_specs=[
          pl.BlockSpec((_BR, 128), lambda i: (i, 0)),
          pl.BlockSpec((_BR, 128), lambda i: (i, 0)),
      ],
      out_shape=[jax.ShapeDtypeStruct((N, 128), jnp.float32),
                 jax.ShapeDtypeStruct((N, 128), jnp.float32)],
  )(xw, deg)


def _layer2_body(a0_ref, a1_ref, deg_ref, b1_ref, w2_ref, o0_ref, o1_ref):
  dinv = lax.rsqrt(deg_ref[:, 0:1] + 1.0)
  h0 = jax.nn.relu(a0_ref[...] * dinv + b1_ref[0:1, :128])
  h1 = jax.nn.relu(a1_ref[...] * dinv + b1_ref[0:1, 128:])
  o = lax.dot_general(h0, w2_ref[:128, :], (((1,), (0,)), ((), ())),
                      precision=_HI, preferred_element_type=jnp.float32)
  o += lax.dot_general(h1, w2_ref[128:, :], (((1,), (0,)), ((), ())),
                       precision=_HI, preferred_element_type=jnp.float32)
  o = o * dinv
  o0_ref[...] = o[:, :32]
  o1_ref[...] = o[:, 32:]


def _tc_layer2(a0, a1, deg, b1, w2):
  return pl.pallas_call(
      _layer2_body,
      grid=(N // _BR,),
      in_specs=[
          pl.BlockSpec((_BR, 128), lambda i: (i, 0)),
          pl.BlockSpec((_BR, 128), lambda i: (i, 0)),
          pl.BlockSpec((_BR, 16), lambda i: (i, 0)),
          pl.BlockSpec((1, H), lambda i: (0, 0)),
          pl.BlockSpec((H, C), lambda i: (0, 0)),
      ],
      out_specs=[
          pl.BlockSpec((_BR, 32), lambda i: (i, 0)),
          pl.BlockSpec((_BR, 32), lambda i: (i, 0)),
      ],
      out_shape=[jax.ShapeDtypeStruct((N, 32), jnp.float32),
                 jax.ShapeDtypeStruct((N, 32), jnp.float32)],
  )(a0, a1, deg, b1, w2)


def _finalize_body(a0_ref, a1_ref, deg_ref, b2_ref, logp_ref, hi_ref, lo_ref):
  dinv = lax.rsqrt(deg_ref[:, 0:1] + 1.0)
  o = jnp.concatenate([a0_ref[...], a1_ref[...]], axis=1) * dinv + b2_ref[0:1, :]
  m = jnp.max(o, axis=1, keepdims=True)
  sh = o - m
  lse = jnp.log(jnp.sum(jnp.exp(sh), axis=1, keepdims=True))
  logp_ref[...] = sh - lse
  sq = jnp.sum(o * o, axis=1, keepdims=True)
  on = o * lax.rsqrt(sq)
  hi = on.astype(jnp.bfloat16)
  hi_ref[...] = hi
  lo_ref[...] = (on - hi.astype(jnp.float32)).astype(jnp.bfloat16)


def _tc_finalize(a0, a1, deg, b2):
  return pl.pallas_call(
      _finalize_body,
      grid=(N // _BR,),
      in_specs=[
          pl.BlockSpec((_BR, 32), lambda i: (i, 0)),
          pl.BlockSpec((_BR, 32), lambda i: (i, 0)),
          pl.BlockSpec((_BR, 16), lambda i: (i, 0)),
          pl.BlockSpec((1, C), lambda i: (0, 0)),
      ],
      out_specs=[
          pl.BlockSpec((_BR, C), lambda i: (i, 0)),
          pl.BlockSpec((_BR, C), lambda i: (i, 0)),
          pl.BlockSpec((_BR, C), lambda i: (i, 0)),
      ],
      out_shape=[jax.ShapeDtypeStruct((N, C), jnp.float32),
                 jax.ShapeDtypeStruct((N, C), jnp.bfloat16),
                 jax.ShapeDtypeStruct((N, C), jnp.bfloat16)],
  )(a0, a1, deg, b2)


_BS = 200


def _sim_body(hi_ref, lo_ref, hit_ref, lot_ref, mask_ref, o_ref):
  dn = (((1,), (0,)), ((), ()))
  hi, lo = hi_ref[...], lo_ref[...]
  hit, lot = hit_ref[...], lot_ref[...]
  s = lax.dot_general(hi, hit, dn, preferred_element_type=jnp.float32)
  s += lax.dot_general(hi, lot, dn, preferred_element_type=jnp.float32)
  s += lax.dot_general(lo, hit, dn, preferred_element_type=jnp.float32)
  o_ref[...] = (1.0 - mask_ref[...]) * s


def _tc_sim(on_hi, on_lo, mask):
  return pl.pallas_call(
      _sim_body,
      grid=(N // _BS,),
      in_specs=[
          pl.BlockSpec((_BS, C), lambda i: (i, 0)),
          pl.BlockSpec((_BS, C), lambda i: (i, 0)),
          pl.BlockSpec((C, N), lambda i: (0, 0)),
          pl.BlockSpec((C, N), lambda i: (0, 0)),
          pl.BlockSpec((_BS, N), lambda i: (i, 0)),
      ],
      out_specs=pl.BlockSpec((_BS, N), lambda i: (i, 0)),
      out_shape=jax.ShapeDtypeStruct((N, N), jnp.float32),
  )(on_hi, on_lo, on_hi.T, on_lo.T, mask)


def kernel(x, edge_index, mask, W1, b1, W2, b2):
  src = edge_index[0].astype(jnp.int32)
  dst = edge_index[1].astype(jnp.int32)

  extra = EPAD - E
  pad_ids = jnp.arange(extra, dtype=jnp.int32)
  src_p = jnp.concatenate([src, pad_ids % N]).reshape(EROWS, 128)
  dst_p = jnp.concatenate([dst, N + pad_ids % (NRA - N)]).reshape(EROWS, 128)

  ones2d = jnp.ones((128, 16), jnp.float32)
  zeros2d = jnp.zeros((NRD, 16), jnp.float32)
  deg16 = _sc_degree(dst_p, ones2d, zeros2d)[:N]

  xw = _tc_matmul1(x, W1)
  hs1_0, hs1_1 = _tc_scale(xw, deg16)
  a1_0, a1_1 = _make_sc_agg(128)(hs1_0, hs1_1, src_p, dst_p)
  hs2_0, hs2_1 = _tc_layer2(a1_0, a1_1, deg16, b1.reshape(1, H), W2)
  a2_0, a2_1 = _make_sc_agg(32)(hs2_0, hs2_1, src_p, dst_p)
  logp, on_hi, on_lo = _tc_finalize(a2_0, a2_1, deg16, b2.reshape(1, C))
  x_dis = _tc_sim(on_hi, on_lo, mask)
  return (logp, x_dis)


# speedup vs baseline: 10.4622x; 1.0228x over previous
import functools

import jax
import jax.numpy as jnp
from jax import lax
from jax.experimental import pallas as pl
from jax.experimental.pallas import tpu as pltpu
from jax.experimental.pallas import tpu_sc as plsc

N = 10000
E = 160000
D = 256
H = 256
C = 64

NSUB = 16
EPAD = 163840
EROWS = EPAD // 128
ROWS_PER_SUB = EROWS // NSUB
CHUNK_ROWS = 8
NCHUNK = ROWS_PER_SUB // CHUNK_ROWS
NRD = 10112
NRA = 10016
IDXROWS = ROWS_PER_SUB // 2
WB = 624
WBTAIL = N - WB * NSUB
DEGWB = NRD // NSUB

_HI = jax.lax.Precision.HIGHEST

@functools.lru_cache(maxsize=None)
def _vector_mesh():
  return plsc.VectorSubcoreMesh(
      core_axis_name="c", subcore_axis_name="s", num_cores=2, num_subcores=NSUB)


def _sc_degree(dst2d, ones2d, zeros2d):
  @functools.partial(
      pl.kernel,
      out_type=jax.ShapeDtypeStruct((NRD, 16), jnp.float32),
      mesh=_vector_mesh(),
      compiler_params=pltpu.CompilerParams(use_tc_tiling_on_sc=False),
      scratch_types=[
          pltpu.VMEM((CHUNK_ROWS, 128), jnp.int32),
          pltpu.VMEM((128, 16), jnp.float32),
          pltpu.VMEM_SHARED((NRD, 16), jnp.float32),
      ],
  )
  def deg_kernel(dst_hbm, ones_hbm, zeros_hbm, out_hbm, dst_v, ones_v, acc_sh):
    c = lax.axis_index("c")
    s = lax.axis_index("s")

    @pl.when(c == 0)
    def _():
      pltpu.sync_copy(ones_hbm, ones_v)
      pltpu.sync_copy(zeros_hbm.at[pl.ds(s * DEGWB, DEGWB)],
                      acc_sh.at[pl.ds(s * DEGWB, DEGWB)])
      plsc.subcore_barrier()

      base = s * ROWS_PER_SUB

      @pl.loop(0, NCHUNK)
      def _(k):
        pltpu.sync_copy(dst_hbm.at[pl.ds(base + k * CHUNK_ROWS, CHUNK_ROWS)],
                        dst_v)

        @pl.loop(0, CHUNK_ROWS)
        def _(j):
          pltpu.sync_copy(ones_v, acc_sh.at[dst_v.at[j]], add=True)

      plsc.subcore_barrier()
      pltpu.sync_copy(acc_sh.at[pl.ds(s * DEGWB, DEGWB)],
                      out_hbm.at[pl.ds(s * DEGWB, DEGWB)])

  return deg_kernel(dst2d, ones2d, zeros2d)


def _make_sc_agg(fc):
  cp = (None if fc % 128 == 0
        else pltpu.CompilerParams(use_tc_tiling_on_sc=False))

  @functools.partial(
      pl.kernel,
      out_type=(jax.ShapeDtypeStruct((N, fc), jnp.float32),
                jax.ShapeDtypeStruct((N, fc), jnp.float32)),
      mesh=_vector_mesh(),
      compiler_params=cp,
      scratch_types=[
          pltpu.VMEM((IDXROWS, 128), jnp.int32),
          pltpu.VMEM((IDXROWS, 128), jnp.int32),
          pltpu.VMEM((128, fc), jnp.float32),
          pltpu.VMEM((128, fc), jnp.float32),
          pltpu.VMEM_SHARED((NRA, fc), jnp.float32),
          pltpu.SemaphoreType.DMA,
          pltpu.SemaphoreType.DMA,
      ],
  )
  def agg_kernel(h0_hbm, h1_hbm, src_hbm, dst_hbm, out0_hbm, out1_hbm,
                 src_v, dst_v, rows_a, rows_b, acc_sh, sem_a, sem_b):
    c = lax.axis_index("c")
    s = lax.axis_index("s")

    def run(h_hbm, out_hbm):
      pltpu.sync_copy(h_hbm.at[pl.ds(s * WB, WB)], acc_sh.at[pl.ds(s * WB, WB)])

      @pl.when(s == 0)
      def _():
        pltpu.sync_copy(h_hbm.at[pl.ds(WB * NSUB, WBTAIL)],
                        acc_sh.at[pl.ds(WB * NSUB, WBTAIL)])

      plsc.subcore_barrier()

      def start(r, buf, sem):
        pltpu.async_copy(h_hbm.at[src_v.at[r]], buf, sem)

      def wait(buf, sem):
        pltpu.make_async_copy(h_hbm.at[pl.ds(0, 128)], buf, sem).wait()

      def scat(r, buf):
        pltpu.sync_copy(buf, acc_sh.at[dst_v.at[r]], add=True)

      @pl.loop(0, 2)
      def _(p):
        base = s * ROWS_PER_SUB + p * IDXROWS
        pltpu.sync_copy(src_hbm.at[pl.ds(base, IDXROWS)], src_v)
        pltpu.sync_copy(dst_hbm.at[pl.ds(base, IDXROWS)], dst_v)
        start(0, rows_a, sem_a)

        @pl.loop(0, IDXROWS, step=2)
        def _(r):
          start(r + 1, rows_b, sem_b)
          wait(rows_a, sem_a)
          scat(r, rows_a)

          @pl.when(r + 2 < IDXROWS)
          def _():
            start(r + 2, rows_a, sem_a)

          wait(rows_b, sem_b)
          scat(r + 1, rows_b)

      plsc.subcore_barrier()
      pltpu.sync_copy(acc_sh.at[pl.ds(s * WB, WB)], out_hbm.at[pl.ds(s * WB, WB)])

      @pl.when(s == 0)
      def _():
        pltpu.sync_copy(acc_sh.at[pl.ds(WB * NSUB, WBTAIL)],
                        out_hbm.at[pl.ds(WB * NSUB, WBTAIL)])

    @pl.when(c == 0)
    def _():
      run(h0_hbm, out0_hbm)

    @pl.when(c == 1)
    def _():
      run(h1_hbm, out1_hbm)

  return agg_kernel


_make_sc_agg = functools.lru_cache(maxsize=None)(_make_sc_agg)


@functools.lru_cache(maxsize=None)
def _make_sc_agg_edges():
  @functools.partial(
      pl.kernel,
      out_type=(jax.ShapeDtypeStruct((N, C), jnp.float32),
                jax.ShapeDtypeStruct((N, C), jnp.float32)),
      mesh=_vector_mesh(),
      compiler_params=pltpu.CompilerParams(use_tc_tiling_on_sc=False),
      scratch_types=[
          pltpu.VMEM((IDXROWS, 128), jnp.int32),
          pltpu.VMEM((IDXROWS, 128), jnp.int32),
          pltpu.VMEM((128, C), jnp.float32),
          pltpu.VMEM((128, C), jnp.float32),
          pltpu.VMEM_SHARED((NRA, C), jnp.float32),
          pltpu.SemaphoreType.DMA,
          pltpu.SemaphoreType.DMA,
      ],
  )
  def agg_kernel(h_hbm, src_hbm, dst_hbm, oa_hbm, ob_hbm,
                 src_v, dst_v, rows_a, rows_b, acc_sh, sem_a, sem_b):
    c = lax.axis_index("c")
    s = lax.axis_index("s")

    pltpu.sync_copy(h_hbm.at[pl.ds(s * WB, WB)], acc_sh.at[pl.ds(s * WB, WB)])

    @pl.when(s == 0)
    def _():
      pltpu.sync_copy(h_hbm.at[pl.ds(WB * NSUB, WBTAIL)],
                      acc_sh.at[pl.ds(WB * NSUB, WBTAIL)])

    plsc.subcore_barrier()

    def start(r, buf, sem):
      pltpu.async_copy(h_hbm.at[src_v.at[r]], buf, sem)

    def wait(buf, sem):
      pltpu.make_async_copy(h_hbm.at[pl.ds(0, 128)], buf, sem).wait()

    def scat(r, buf):
      pltpu.sync_copy(buf, acc_sh.at[dst_v.at[r]], add=True)

    base = c * (EROWS // 2) + s * IDXROWS
    pltpu.sync_copy(src_hbm.at[pl.ds(base, IDXROWS)], src_v)
    pltpu.sync_copy(dst_hbm.at[pl.ds(base, IDXROWS)], dst_v)
    start(0, rows_a, sem_a)

    @pl.loop(0, IDXROWS, step=2)
    def _(r):
      start(r + 1, rows_b, sem_b)
      wait(rows_a, sem_a)
      scat(r, rows_a)

      @pl.when(r + 2 < IDXROWS)
      def _():
        start(r + 2, rows_a, sem_a)

      wait(rows_b, sem_b)
      scat(r + 1, rows_b)

    plsc.subcore_barrier()

    def wb(out_hbm):
      pltpu.sync_copy(acc_sh.at[pl.ds(s * WB, WB)],
                      out_hbm.at[pl.ds(s * WB, WB)])

      @pl.when(s == 0)
      def _():
        pltpu.sync_copy(acc_sh.at[pl.ds(WB * NSUB, WBTAIL)],
                        out_hbm.at[pl.ds(WB * NSUB, WBTAIL)])

    @pl.when(c == 0)
    def _():
      wb(oa_hbm)

    @pl.when(c == 1)
    def _():
      wb(ob_hbm)

  return agg_kernel


_BR = 1000


def _mm1_body(x_ref, w_ref, deg_ref, o0_ref, o1_ref):
  dinv = lax.rsqrt(deg_ref[:, 0:1] + 1.0)
  o = lax.dot_general(x_ref[...], w_ref[...], (((1,), (0,)), ((), ())),
                      precision=_HI, preferred_element_type=jnp.float32)
  o = o * dinv
  o0_ref[...] = o[:, :128]
  o1_ref[...] = o[:, 128:]


def _tc_mm1scale(x, w1, deg):
  return pl.pallas_call(
      _mm1_body,
      grid=(N // _BR,),
      in_specs=[
          pl.BlockSpec((_BR, D), lambda i: (i, 0)),
          pl.BlockSpec((D, H), lambda i: (0, 0)),
          pl.BlockSpec((_BR, 16), lambda i: (i, 0)),
      ],
      out_specs=[
          pl.BlockSpec((_BR, 128), lambda i: (i, 0)),
          pl.BlockSpec((_BR, 128), lambda i: (i, 0)),
      ],
      out_shape=[jax.ShapeDtypeStruct((N, 128), jnp.float32),
                 jax.ShapeDtypeStruct((N, 128), jnp.float32)],
  )(x, w1, deg)


def _layer2_body(a0_ref, a1_ref, deg_ref, b1_ref, w2_ref, o_ref):
  dinv = lax.rsqrt(deg_ref[:, 0:1] + 1.0)
  h0 = jax.nn.relu(a0_ref[...] * dinv + b1_ref[0:1, :128])
  h1 = jax.nn.relu(a1_ref[...] * dinv + b1_ref[0:1, 128:])
  o = lax.dot_general(h0, w2_ref[:128, :], (((1,), (0,)), ((), ())),
                      precision=_HI, preferred_element_type=jnp.float32)
  o += lax.dot_general(h1, w2_ref[128:, :], (((1,), (0,)), ((), ())),
                       precision=_HI, preferred_element_type=jnp.float32)
  o_ref[...] = o * dinv


def _tc_layer2(a0, a1, deg, b1, w2):
  return pl.pallas_call(
      _layer2_body,
      grid=(N // _BR,),
      in_specs=[
          pl.BlockSpec((_BR, 128), lambda i: (i, 0)),
          pl.BlockSpec((_BR, 128), lambda i: (i, 0)),
          pl.BlockSpec((_BR, 16), lambda i: (i, 0)),
          pl.BlockSpec((1, H), lambda i: (0, 0)),
          pl.BlockSpec((H, C), lambda i: (0, 0)),
      ],
      out_specs=pl.BlockSpec((_BR, C), lambda i: (i, 0)),
      out_shape=jax.ShapeDtypeStruct((N, C), jnp.float32),
  )(a0, a1, deg, b1, w2)


def _finalize_body(aa_ref, ab_ref, hs_ref, deg_ref, b2_ref,
                   logp_ref, hi_ref, lo_ref):
  dinv = lax.rsqrt(deg_ref[:, 0:1] + 1.0)
  o = (aa_ref[...] + ab_ref[...] - hs_ref[...]) * dinv + b2_ref[0:1, :]
  m = jnp.max(o, axis=1, keepdims=True)
  sh = o - m
  lse = jnp.log(jnp.sum(jnp.exp(sh), axis=1, keepdims=True))
  logp_ref[...] = sh - lse
  sq = jnp.sum(o * o, axis=1, keepdims=True)
  on = o * lax.rsqrt(sq)
  hi = on.astype(jnp.bfloat16)
  hi_ref[...] = hi
  lo_ref[...] = (on - hi.astype(jnp.float32)).astype(jnp.bfloat16)


def _tc_finalize(aa, ab, hs2, deg, b2):
  return pl.pallas_call(
      _finalize_body,
      grid=(N // _BR,),
      in_specs=[
          pl.BlockSpec((_BR, C), lambda i: (i, 0)),
          pl.BlockSpec((_BR, C), lambda i: (i, 0)),
          pl.BlockSpec((_BR, C), lambda i: (i, 0)),
          pl.BlockSpec((_BR, 16), lambda i: (i, 0)),
          pl.BlockSpec((1, C), lambda i: (0, 0)),
      ],
      out_specs=[
          pl.BlockSpec((_BR, C), lambda i: (i, 0)),
          pl.BlockSpec((_BR, C), lambda i: (i, 0)),
          pl.BlockSpec((_BR, C), lambda i: (i, 0)),
      ],
      out_shape=[jax.ShapeDtypeStruct((N, C), jnp.float32),
                 jax.ShapeDtypeStruct((N, C), jnp.bfloat16),
                 jax.ShapeDtypeStruct((N, C), jnp.bfloat16)],
  )(aa, ab, hs2, deg, b2)


_BS = 200


def _sim_body(hi_ref, lo_ref, hit_ref, lot_ref, mask_ref, o_ref):
  dn = (((1,), (0,)), ((), ()))
  hi, lo = hi_ref[...], lo_ref[...]
  hit, lot = hit_ref[...], lot_ref[...]
  s = lax.dot_general(hi, hit, dn, preferred_element_type=jnp.float32)
  s += lax.dot_general(hi, lot, dn, preferred_element_type=jnp.float32)
  s += lax.dot_general(lo, hit, dn, preferred_element_type=jnp.float32)
  o_ref[...] = (1.0 - mask_ref[...]) * s


def _tc_sim(on_hi, on_lo, mask):
  return pl.pallas_call(
      _sim_body,
      grid=(N // _BS,),
      in_specs=[
          pl.BlockSpec((_BS, C), lambda i: (i, 0)),
          pl.BlockSpec((_BS, C), lambda i: (i, 0)),
          pl.BlockSpec((C, N), lambda i: (0, 0)),
          pl.BlockSpec((C, N), lambda i: (0, 0)),
          pl.BlockSpec((_BS, N), lambda i: (i, 0)),
      ],
      out_specs=pl.BlockSpec((_BS, N), lambda i: (i, 0)),
      out_shape=jax.ShapeDtypeStruct((N, N), jnp.float32),
  )(on_hi, on_lo, on_hi.T, on_lo.T, mask)


def kernel(x, edge_index, mask, W1, b1, W2, b2):
  src = edge_index[0].astype(jnp.int32)
  dst = edge_index[1].astype(jnp.int32)

  extra = EPAD - E
  pad_ids = jnp.arange(extra, dtype=jnp.int32)
  src_p = jnp.concatenate([src, pad_ids % N]).reshape(EROWS, 128)
  dst_p = jnp.concatenate([dst, N + pad_ids % (NRA - N)]).reshape(EROWS, 128)

  ones2d = jnp.ones((128, 16), jnp.float32)
  zeros2d = jnp.zeros((NRD, 16), jnp.float32)
  deg16 = _sc_degree(dst_p, ones2d, zeros2d)[:N]

  hs1_0, hs1_1 = _tc_mm1scale(x, W1, deg16)
  a1_0, a1_1 = _make_sc_agg(128)(hs1_0, hs1_1, src_p, dst_p)
  hs2 = _tc_layer2(a1_0, a1_1, deg16, b1.reshape(1, H), W2)
  a2_a, a2_b = _make_sc_agg_edges()(hs2, src_p, dst_p)
  logp, on_hi, on_lo = _tc_finalize(a2_a, a2_b, hs2, deg16, b2.reshape(1, C))
  x_dis = _tc_sim(on_hi, on_lo, mask)
  return (logp, x_dis)


# speedup vs baseline: 10.5551x; 1.0089x over previous
import functools

import jax
import jax.numpy as jnp
from jax import lax
from jax.experimental import pallas as pl
from jax.experimental.pallas import tpu as pltpu
from jax.experimental.pallas import tpu_sc as plsc

N = 10000
E = 160000
D = 256
H = 256
C = 64

NSUB = 16
EPAD = 163840
EROWS = EPAD // 128
ROWS_PER_SUB = EROWS // NSUB
CHUNK_ROWS = 8
NCHUNK = ROWS_PER_SUB // CHUNK_ROWS
NRD = 10112
NRA = 10016
IDXROWS = ROWS_PER_SUB // 2
WB = 624
WBTAIL = N - WB * NSUB
DEGWB = NRD // NSUB

_HI = jax.lax.Precision.HIGHEST

@functools.lru_cache(maxsize=None)
def _vector_mesh():
  return plsc.VectorSubcoreMesh(
      core_axis_name="c", subcore_axis_name="s", num_cores=2, num_subcores=NSUB)


def _sc_degree(dst2d, ones2d, zeros2d):
  @functools.partial(
      pl.kernel,
      out_type=(jax.ShapeDtypeStruct((NRD, 16), jnp.float32),
                jax.ShapeDtypeStruct((NRD, 16), jnp.float32)),
      mesh=_vector_mesh(),
      compiler_params=pltpu.CompilerParams(use_tc_tiling_on_sc=False),
      scratch_types=[
          pltpu.VMEM((IDXROWS, 128), jnp.int32),
          pltpu.VMEM((128, 16), jnp.float32),
          pltpu.VMEM_SHARED((NRD, 16), jnp.float32),
      ],
  )
  def deg_kernel(dst_hbm, ones_hbm, zeros_hbm, outa_hbm, outb_hbm,
                 dst_v, ones_v, acc_sh):
    c = lax.axis_index("c")
    s = lax.axis_index("s")

    pltpu.sync_copy(ones_hbm, ones_v)
    pltpu.sync_copy(zeros_hbm.at[pl.ds(s * DEGWB, DEGWB)],
                    acc_sh.at[pl.ds(s * DEGWB, DEGWB)])
    plsc.subcore_barrier()

    base = c * (EROWS // 2) + s * IDXROWS
    pltpu.sync_copy(dst_hbm.at[pl.ds(base, IDXROWS)], dst_v)

    @pl.loop(0, IDXROWS)
    def _(j):
      pltpu.sync_copy(ones_v, acc_sh.at[dst_v.at[j]], add=True)

    plsc.subcore_barrier()

    def wb(out_hbm):
      pltpu.sync_copy(acc_sh.at[pl.ds(s * DEGWB, DEGWB)],
                      out_hbm.at[pl.ds(s * DEGWB, DEGWB)])

    @pl.when(c == 0)
    def _():
      wb(outa_hbm)

    @pl.when(c == 1)
    def _():
      wb(outb_hbm)

  return deg_kernel(dst2d, ones2d, zeros2d)


def _make_sc_agg(fc):
  cp = (None if fc % 128 == 0
        else pltpu.CompilerParams(use_tc_tiling_on_sc=False))

  @functools.partial(
      pl.kernel,
      out_type=(jax.ShapeDtypeStruct((N, fc), jnp.float32),
                jax.ShapeDtypeStruct((N, fc), jnp.float32)),
      mesh=_vector_mesh(),
      compiler_params=cp,
      scratch_types=[
          pltpu.VMEM((IDXROWS, 128), jnp.int32),
          pltpu.VMEM((IDXROWS, 128), jnp.int32),
          pltpu.VMEM((128, fc), jnp.float32),
          pltpu.VMEM((128, fc), jnp.float32),
          pltpu.VMEM_SHARED((NRA, fc), jnp.float32),
          pltpu.SemaphoreType.DMA,
          pltpu.SemaphoreType.DMA,
      ],
  )
  def agg_kernel(h0_hbm, h1_hbm, src_hbm, dst_hbm, out0_hbm, out1_hbm,
                 src_v, dst_v, rows_a, rows_b, acc_sh, sem_a, sem_b):
    c = lax.axis_index("c")
    s = lax.axis_index("s")

    def run(h_hbm, out_hbm):
      pltpu.sync_copy(h_hbm.at[pl.ds(s * WB, WB)], acc_sh.at[pl.ds(s * WB, WB)])

      @pl.when(s == 0)
      def _():
        pltpu.sync_copy(h_hbm.at[pl.ds(WB * NSUB, WBTAIL)],
                        acc_sh.at[pl.ds(WB * NSUB, WBTAIL)])

      plsc.subcore_barrier()

      def start(r, buf, sem):
        pltpu.async_copy(h_hbm.at[src_v.at[r]], buf, sem)

      def wait(buf, sem):
        pltpu.make_async_copy(h_hbm.at[pl.ds(0, 128)], buf, sem).wait()

      def scat(r, buf):
        pltpu.sync_copy(buf, acc_sh.at[dst_v.at[r]], add=True)

      @pl.loop(0, 2)
      def _(p):
        base = s * ROWS_PER_SUB + p * IDXROWS
        pltpu.sync_copy(src_hbm.at[pl.ds(base, IDXROWS)], src_v)
        pltpu.sync_copy(dst_hbm.at[pl.ds(base, IDXROWS)], dst_v)
        start(0, rows_a, sem_a)

        @pl.loop(0, IDXROWS, step=2)
        def _(r):
          start(r + 1, rows_b, sem_b)
          wait(rows_a, sem_a)
          scat(r, rows_a)

          @pl.when(r + 2 < IDXROWS)
          def _():
            start(r + 2, rows_a, sem_a)

          wait(rows_b, sem_b)
          scat(r + 1, rows_b)

      plsc.subcore_barrier()
      pltpu.sync_copy(acc_sh.at[pl.ds(s * WB, WB)], out_hbm.at[pl.ds(s * WB, WB)])

      @pl.when(s == 0)
      def _():
        pltpu.sync_copy(acc_sh.at[pl.ds(WB * NSUB, WBTAIL)],
                        out_hbm.at[pl.ds(WB * NSUB, WBTAIL)])

    @pl.when(c == 0)
    def _():
      run(h0_hbm, out0_hbm)

    @pl.when(c == 1)
    def _():
      run(h1_hbm, out1_hbm)

  return agg_kernel


_make_sc_agg = functools.lru_cache(maxsize=None)(_make_sc_agg)


@functools.lru_cache(maxsize=None)
def _make_sc_agg_edges():
  @functools.partial(
      pl.kernel,
      out_type=(jax.ShapeDtypeStruct((N, C), jnp.float32),
                jax.ShapeDtypeStruct((N, C), jnp.float32)),
      mesh=_vector_mesh(),
      compiler_params=pltpu.CompilerParams(use_tc_tiling_on_sc=False),
      scratch_types=[
          pltpu.VMEM((IDXROWS, 128), jnp.int32),
          pltpu.VMEM((IDXROWS, 128), jnp.int32),
          pltpu.VMEM((128, C), jnp.float32),
          pltpu.VMEM((128, C), jnp.float32),
          pltpu.VMEM_SHARED((NRA, C), jnp.float32),
          pltpu.SemaphoreType.DMA,
          pltpu.SemaphoreType.DMA,
      ],
  )
  def agg_kernel(h_hbm, src_hbm, dst_hbm, oa_hbm, ob_hbm,
                 src_v, dst_v, rows_a, rows_b, acc_sh, sem_a, sem_b):
    c = lax.axis_index("c")
    s = lax.axis_index("s")

    pltpu.sync_copy(h_hbm.at[pl.ds(s * WB, WB)], acc_sh.at[pl.ds(s * WB, WB)])

    @pl.when(s == 0)
    def _():
      pltpu.sync_copy(h_hbm.at[pl.ds(WB * NSUB, WBTAIL)],
                      acc_sh.at[pl.ds(WB * NSUB, WBTAIL)])

    plsc.subcore_barrier()

    def start(r, buf, sem):
      pltpu.async_copy(h_hbm.at[src_v.at[r]], buf, sem)

    def wait(buf, sem):
      pltpu.make_async_copy(h_hbm.at[pl.ds(0, 128)], buf, sem).wait()

    def scat(r, buf):
      pltpu.sync_copy(buf, acc_sh.at[dst_v.at[r]], add=True)

    base = c * (EROWS // 2) + s * IDXROWS
    pltpu.sync_copy(src_hbm.at[pl.ds(base, IDXROWS)], src_v)
    pltpu.sync_copy(dst_hbm.at[pl.ds(base, IDXROWS)], dst_v)
    start(0, rows_a, sem_a)

    @pl.loop(0, IDXROWS, step=2)
    def _(r):
      start(r + 1, rows_b, sem_b)
      wait(rows_a, sem_a)
      scat(r, rows_a)

      @pl.when(r + 2 < IDXROWS)
      def _():
        start(r + 2, rows_a, sem_a)

      wait(rows_b, sem_b)
      scat(r + 1, rows_b)

    plsc.subcore_barrier()

    def wb(out_hbm):
      pltpu.sync_copy(acc_sh.at[pl.ds(s * WB, WB)],
                      out_hbm.at[pl.ds(s * WB, WB)])

      @pl.when(s == 0)
      def _():
        pltpu.sync_copy(acc_sh.at[pl.ds(WB * NSUB, WBTAIL)],
                        out_hbm.at[pl.ds(WB * NSUB, WBTAIL)])

    @pl.when(c == 0)
    def _():
      wb(oa_hbm)

    @pl.when(c == 1)
    def _():
      wb(ob_hbm)

  return agg_kernel


_BR = 1000


def _mm1_body(x_ref, w_ref, dega_ref, degb_ref, o0_ref, o1_ref):
  dinv = lax.rsqrt(dega_ref[:, 0:1] + degb_ref[:, 0:1] + 1.0)
  o = lax.dot_general(x_ref[...], w_ref[...], (((1,), (0,)), ((), ())),
                      precision=_HI, preferred_element_type=jnp.float32)
  o = o * dinv
  o0_ref[...] = o[:, :128]
  o1_ref[...] = o[:, 128:]


def _tc_mm1scale(x, w1, dega, degb):
  return pl.pallas_call(
      _mm1_body,
      grid=(N // _BR,),
      in_specs=[
          pl.BlockSpec((_BR, D), lambda i: (i, 0)),
          pl.BlockSpec((D, H), lambda i: (0, 0)),
          pl.BlockSpec((_BR, 16), lambda i: (i, 0)),
          pl.BlockSpec((_BR, 16), lambda i: (i, 0)),
      ],
      out_specs=[
          pl.BlockSpec((_BR, 128), lambda i: (i, 0)),
          pl.BlockSpec((_BR, 128), lambda i: (i, 0)),
      ],
      out_shape=[jax.ShapeDtypeStruct((N, 128), jnp.float32),
                 jax.ShapeDtypeStruct((N, 128), jnp.float32)],
  )(x, w1, dega, degb)


def _layer2_body(a0_ref, a1_ref, dega_ref, degb_ref, b1_ref, w2_ref, o_ref):
  dinv = lax.rsqrt(dega_ref[:, 0:1] + degb_ref[:, 0:1] + 1.0)
  h0 = jax.nn.relu(a0_ref[...] * dinv + b1_ref[0:1, :128])
  h1 = jax.nn.relu(a1_ref[...] * dinv + b1_ref[0:1, 128:])
  o = lax.dot_general(h0, w2_ref[:128, :], (((1,), (0,)), ((), ())),
                      precision=_HI, preferred_element_type=jnp.float32)
  o += lax.dot_general(h1, w2_ref[128:, :], (((1,), (0,)), ((), ())),
                       precision=_HI, preferred_element_type=jnp.float32)
  o_ref[...] = o * dinv


def _tc_layer2(a0, a1, dega, degb, b1, w2):
  return pl.pallas_call(
      _layer2_body,
      grid=(N // _BR,),
      in_specs=[
          pl.BlockSpec((_BR, 128), lambda i: (i, 0)),
          pl.BlockSpec((_BR, 128), lambda i: (i, 0)),
          pl.BlockSpec((_BR, 16), lambda i: (i, 0)),
          pl.BlockSpec((_BR, 16), lambda i: (i, 0)),
          pl.BlockSpec((1, H), lambda i: (0, 0)),
          pl.BlockSpec((H, C), lambda i: (0, 0)),
      ],
      out_specs=pl.BlockSpec((_BR, C), lambda i: (i, 0)),
      out_shape=jax.ShapeDtypeStruct((N, C), jnp.float32),
  )(a0, a1, dega, degb, b1, w2)


def _finalize_body(aa_ref, ab_ref, hs_ref, dega_ref, degb_ref, b2_ref,
                   logp_ref, hi_ref, lo_ref):
  dinv = lax.rsqrt(dega_ref[:, 0:1] + degb_ref[:, 0:1] + 1.0)
  o = (aa_ref[...] + ab_ref[...] - hs_ref[...]) * dinv + b2_ref[0:1, :]
  m = jnp.max(o, axis=1, keepdims=True)
  sh = o - m
  lse = jnp.log(jnp.sum(jnp.exp(sh), axis=1, keepdims=True))
  logp_ref[...] = sh - lse
  sq = jnp.sum(o * o, axis=1, keepdims=True)
  on = o * lax.rsqrt(sq)
  hi = on.astype(jnp.bfloat16)
  hi_ref[...] = hi
  lo_ref[...] = (on - hi.astype(jnp.float32)).astype(jnp.bfloat16)


def _tc_finalize(aa, ab, hs2, dega, degb, b2):
  return pl.pallas_call(
      _finalize_body,
      grid=(N // _BR,),
      in_specs=[
          pl.BlockSpec((_BR, C), lambda i: (i, 0)),
          pl.BlockSpec((_BR, C), lambda i: (i, 0)),
          pl.BlockSpec((_BR, C), lambda i: (i, 0)),
          pl.BlockSpec((_BR, 16), lambda i: (i, 0)),
          pl.BlockSpec((_BR, 16), lambda i: (i, 0)),
          pl.BlockSpec((1, C), lambda i: (0, 0)),
      ],
      out_specs=[
          pl.BlockSpec((_BR, C), lambda i: (i, 0)),
          pl.BlockSpec((_BR, C), lambda i: (i, 0)),
          pl.BlockSpec((_BR, C), lambda i: (i, 0)),
      ],
      out_shape=[jax.ShapeDtypeStruct((N, C), jnp.float32),
                 jax.ShapeDtypeStruct((N, C), jnp.bfloat16),
                 jax.ShapeDtypeStruct((N, C), jnp.bfloat16)],
  )(aa, ab, hs2, dega, degb, b2)


_BS = 200


def _sim_body(hi_ref, lo_ref, hit_ref, lot_ref, mask_ref, o_ref):
  dn = (((1,), (0,)), ((), ()))
  hi, lo = hi_ref[...], lo_ref[...]
  hit, lot = hit_ref[...], lot_ref[...]
  s = lax.dot_general(hi, hit, dn, preferred_element_type=jnp.float32)
  s += lax.dot_general(hi, lot, dn, preferred_element_type=jnp.float32)
  s += lax.dot_general(lo, hit, dn, preferred_element_type=jnp.float32)
  o_ref[...] = (1.0 - mask_ref[...]) * s


def _tc_sim(on_hi, on_lo, mask):
  return pl.pallas_call(
      _sim_body,
      grid=(N // _BS,),
      in_specs=[
          pl.BlockSpec((_BS, C), lambda i: (i, 0)),
          pl.BlockSpec((_BS, C), lambda i: (i, 0)),
          pl.BlockSpec((C, N), lambda i: (0, 0)),
          pl.BlockSpec((C, N), lambda i: (0, 0)),
          pl.BlockSpec((_BS, N), lambda i: (i, 0)),
      ],
      out_specs=pl.BlockSpec((_BS, N), lambda i: (i, 0)),
      out_shape=jax.ShapeDtypeStruct((N, N), jnp.float32),
  )(on_hi, on_lo, on_hi.T, on_lo.T, mask)


def kernel(x, edge_index, mask, W1, b1, W2, b2):
  src = edge_index[0].astype(jnp.int32)
  dst = edge_index[1].astype(jnp.int32)

  extra = EPAD - E
  pad_ids = jnp.arange(extra, dtype=jnp.int32)
  src_p = jnp.concatenate([src, pad_ids % N]).reshape(EROWS, 128)
  dst_p = jnp.concatenate([dst, N + pad_ids % (NRA - N)]).reshape(EROWS, 128)

  ones2d = jnp.ones((128, 16), jnp.float32)
  zeros2d = jnp.zeros((NRD, 16), jnp.float32)
  dega, degb = _sc_degree(dst_p, ones2d, zeros2d)

  hs1_0, hs1_1 = _tc_mm1scale(x, W1, dega, degb)
  a1_0, a1_1 = _make_sc_agg(128)(hs1_0, hs1_1, src_p, dst_p)
  hs2 = _tc_layer2(a1_0, a1_1, dega, degb, b1.reshape(1, H), W2)
  a2_a, a2_b = _make_sc_agg_edges()(hs2, src_p, dst_p)
  logp, on_hi, on_lo = _tc_finalize(a2_a, a2_b, hs2, dega, degb,
                                    b2.reshape(1, C))
  x_dis = _tc_sim(on_hi, on_lo, mask)
  return (logp, x_dis)


# speedup vs baseline: 10.5705x; 1.0015x over previous
import functools

import jax
import jax.numpy as jnp
from jax import lax
from jax.experimental import pallas as pl
from jax.experimental.pallas import tpu as pltpu
from jax.experimental.pallas import tpu_sc as plsc

N = 10000
E = 160000
D = 256
H = 256
C = 64

NSUB = 16
EPAD = 163840
EROWS = EPAD // 128
ROWS_PER_SUB = EROWS // NSUB
CHUNK_ROWS = 8
NCHUNK = ROWS_PER_SUB // CHUNK_ROWS
NRD = 10112
NRA = 10016
IDXROWS = ROWS_PER_SUB // 2
WB = 624
WBTAIL = N - WB * NSUB
DEGWB = NRD // NSUB

_HI = jax.lax.Precision.HIGHEST

@functools.lru_cache(maxsize=None)
def _vector_mesh():
  return plsc.VectorSubcoreMesh(
      core_axis_name="c", subcore_axis_name="s", num_cores=2, num_subcores=NSUB)


def _sc_degree(dst2d, ones2d, zeros2d):
  @functools.partial(
      pl.kernel,
      out_type=(jax.ShapeDtypeStruct((NRD, 16), jnp.float32),
                jax.ShapeDtypeStruct((NRD, 16), jnp.float32)),
      mesh=_vector_mesh(),
      compiler_params=pltpu.CompilerParams(use_tc_tiling_on_sc=False),
      scratch_types=[
          pltpu.VMEM((IDXROWS, 128), jnp.int32),
          pltpu.VMEM((128, 16), jnp.float32),
          pltpu.VMEM_SHARED((NRD, 16), jnp.float32),
      ],
  )
  def deg_kernel(dst_hbm, ones_hbm, zeros_hbm, outa_hbm, outb_hbm,
                 dst_v, ones_v, acc_sh):
    c = lax.axis_index("c")
    s = lax.axis_index("s")

    pltpu.sync_copy(ones_hbm, ones_v)
    pltpu.sync_copy(zeros_hbm.at[pl.ds(s * DEGWB, DEGWB)],
                    acc_sh.at[pl.ds(s * DEGWB, DEGWB)])
    plsc.subcore_barrier()

    base = c * (EROWS // 2) + s * IDXROWS
    pltpu.sync_copy(dst_hbm.at[pl.ds(base, IDXROWS)], dst_v)

    @pl.loop(0, IDXROWS)
    def _(j):
      pltpu.sync_copy(ones_v, acc_sh.at[dst_v.at[j]], add=True)

    plsc.subcore_barrier()

    def wb(out_hbm):
      pltpu.sync_copy(acc_sh.at[pl.ds(s * DEGWB, DEGWB)],
                      out_hbm.at[pl.ds(s * DEGWB, DEGWB)])

    @pl.when(c == 0)
    def _():
      wb(outa_hbm)

    @pl.when(c == 1)
    def _():
      wb(outb_hbm)

  return deg_kernel(dst2d, ones2d, zeros2d)


def _make_sc_agg(fc):
  cp = (None if fc % 128 == 0
        else pltpu.CompilerParams(use_tc_tiling_on_sc=False))

  @functools.partial(
      pl.kernel,
      out_type=(jax.ShapeDtypeStruct((N, fc), jnp.float32),
                jax.ShapeDtypeStruct((N, fc), jnp.float32)),
      mesh=_vector_mesh(),
      compiler_params=cp,
      scratch_types=[
          pltpu.VMEM((IDXROWS, 128), jnp.int32),
          pltpu.VMEM((IDXROWS, 128), jnp.int32),
          pltpu.VMEM((128, fc), jnp.float32),
          pltpu.VMEM((128, fc), jnp.float32),
          pltpu.VMEM_SHARED((NRA, fc), jnp.float32),
          pltpu.SemaphoreType.DMA,
          pltpu.SemaphoreType.DMA,
      ],
  )
  def agg_kernel(h0_hbm, h1_hbm, src_hbm, dst_hbm, out0_hbm, out1_hbm,
                 src_v, dst_v, rows_a, rows_b, acc_sh, sem_a, sem_b):
    c = lax.axis_index("c")
    s = lax.axis_index("s")

    def run(h_hbm, out_hbm):
      pltpu.sync_copy(h_hbm.at[pl.ds(s * WB, WB)], acc_sh.at[pl.ds(s * WB, WB)])

      @pl.when(s == 0)
      def _():
        pltpu.sync_copy(h_hbm.at[pl.ds(WB * NSUB, WBTAIL)],
                        acc_sh.at[pl.ds(WB * NSUB, WBTAIL)])

      plsc.subcore_barrier()

      def start(r, buf, sem):
        pltpu.async_copy(h_hbm.at[src_v.at[r]], buf, sem)

      def wait(buf, sem):
        pltpu.make_async_copy(h_hbm.at[pl.ds(0, 128)], buf, sem).wait()

      def scat(r, buf):
        pltpu.sync_copy(buf, acc_sh.at[dst_v.at[r]], add=True)

      @pl.loop(0, 2)
      def _(p):
        base = s * ROWS_PER_SUB + p * IDXROWS
        pltpu.sync_copy(src_hbm.at[pl.ds(base, IDXROWS)], src_v)
        pltpu.sync_copy(dst_hbm.at[pl.ds(base, IDXROWS)], dst_v)
        start(0, rows_a, sem_a)

        @pl.loop(0, IDXROWS, step=2)
        def _(r):
          start(r + 1, rows_b, sem_b)
          wait(rows_a, sem_a)
          scat(r, rows_a)

          @pl.when(r + 2 < IDXROWS)
          def _():
            start(r + 2, rows_a, sem_a)

          wait(rows_b, sem_b)
          scat(r + 1, rows_b)

      plsc.subcore_barrier()
      pltpu.sync_copy(acc_sh.at[pl.ds(s * WB, WB)], out_hbm.at[pl.ds(s * WB, WB)])

      @pl.when(s == 0)
      def _():
        pltpu.sync_copy(acc_sh.at[pl.ds(WB * NSUB, WBTAIL)],
                        out_hbm.at[pl.ds(WB * NSUB, WBTAIL)])

    @pl.when(c == 0)
    def _():
      run(h0_hbm, out0_hbm)

    @pl.when(c == 1)
    def _():
      run(h1_hbm, out1_hbm)

  return agg_kernel


_make_sc_agg = functools.lru_cache(maxsize=None)(_make_sc_agg)


@functools.lru_cache(maxsize=None)
def _make_sc_agg_edges():
  @functools.partial(
      pl.kernel,
      out_type=(jax.ShapeDtypeStruct((N, C), jnp.float32),
                jax.ShapeDtypeStruct((N, C), jnp.float32)),
      mesh=_vector_mesh(),
      compiler_params=pltpu.CompilerParams(use_tc_tiling_on_sc=False),
      scratch_types=[
          pltpu.VMEM((IDXROWS, 128), jnp.int32),
          pltpu.VMEM((IDXROWS, 128), jnp.int32),
          pltpu.VMEM((128, C), jnp.float32),
          pltpu.VMEM((128, C), jnp.float32),
          pltpu.VMEM_SHARED((NRA, C), jnp.float32),
          pltpu.SemaphoreType.DMA,
          pltpu.SemaphoreType.DMA,
      ],
  )
  def agg_kernel(h_hbm, src_hbm, dst_hbm, oa_hbm, ob_hbm,
                 src_v, dst_v, rows_a, rows_b, acc_sh, sem_a, sem_b):
    c = lax.axis_index("c")
    s = lax.axis_index("s")

    pltpu.sync_copy(h_hbm.at[pl.ds(s * WB, WB)], acc_sh.at[pl.ds(s * WB, WB)])

    @pl.when(s == 0)
    def _():
      pltpu.sync_copy(h_hbm.at[pl.ds(WB * NSUB, WBTAIL)],
                      acc_sh.at[pl.ds(WB * NSUB, WBTAIL)])

    plsc.subcore_barrier()

    def start(r, buf, sem):
      pltpu.async_copy(h_hbm.at[src_v.at[r]], buf, sem)

    def wait(buf, sem):
      pltpu.make_async_copy(h_hbm.at[pl.ds(0, 128)], buf, sem).wait()

    def scat(r, buf):
      pltpu.sync_copy(buf, acc_sh.at[dst_v.at[r]], add=True)

    base = c * (EROWS // 2) + s * IDXROWS
    pltpu.sync_copy(src_hbm.at[pl.ds(base, IDXROWS)], src_v)
    pltpu.sync_copy(dst_hbm.at[pl.ds(base, IDXROWS)], dst_v)
    start(0, rows_a, sem_a)

    @pl.loop(0, IDXROWS, step=2)
    def _(r):
      start(r + 1, rows_b, sem_b)
      wait(rows_a, sem_a)
      scat(r, rows_a)

      @pl.when(r + 2 < IDXROWS)
      def _():
        start(r + 2, rows_a, sem_a)

      wait(rows_b, sem_b)
      scat(r + 1, rows_b)

    plsc.subcore_barrier()

    def wb(out_hbm):
      pltpu.sync_copy(acc_sh.at[pl.ds(s * WB, WB)],
                      out_hbm.at[pl.ds(s * WB, WB)])

      @pl.when(s == 0)
      def _():
        pltpu.sync_copy(acc_sh.at[pl.ds(WB * NSUB, WBTAIL)],
                        out_hbm.at[pl.ds(WB * NSUB, WBTAIL)])

    @pl.when(c == 0)
    def _():
      wb(oa_hbm)

    @pl.when(c == 1)
    def _():
      wb(ob_hbm)

  return agg_kernel


_BR = 1000


def _mm1_body(x_ref, w_ref, dega_ref, degb_ref, o0_ref, o1_ref):
  dinv = lax.rsqrt(dega_ref[:, 0:1] + degb_ref[:, 0:1] + 1.0)
  o = lax.dot_general(x_ref[...], w_ref[...], (((1,), (0,)), ((), ())),
                      precision=_HI, preferred_element_type=jnp.float32)
  o = o * dinv
  o0_ref[...] = o[:, :128]
  o1_ref[...] = o[:, 128:]


def _tc_mm1scale(x, w1, dega, degb):
  return pl.pallas_call(
      _mm1_body,
      grid=(N // _BR,),
      in_specs=[
          pl.BlockSpec((_BR, D), lambda i: (i, 0)),
          pl.BlockSpec((D, H), lambda i: (0, 0)),
          pl.BlockSpec((_BR, 16), lambda i: (i, 0)),
          pl.BlockSpec((_BR, 16), lambda i: (i, 0)),
      ],
      out_specs=[
          pl.BlockSpec((_BR, 128), lambda i: (i, 0)),
          pl.BlockSpec((_BR, 128), lambda i: (i, 0)),
      ],
      out_shape=[jax.ShapeDtypeStruct((N, 128), jnp.float32),
                 jax.ShapeDtypeStruct((N, 128), jnp.float32)],
  )(x, w1, dega, degb)


def _layer2_body(a0_ref, a1_ref, dega_ref, degb_ref, b1_ref, w2_ref, o_ref):
  dinv = lax.rsqrt(dega_ref[:, 0:1] + degb_ref[:, 0:1] + 1.0)
  h0 = jax.nn.relu(a0_ref[...] * dinv + b1_ref[0:1, :128])
  h1 = jax.nn.relu(a1_ref[...] * dinv + b1_ref[0:1, 128:])
  o = lax.dot_general(h0, w2_ref[:128, :], (((1,), (0,)), ((), ())),
                      precision=_HI, preferred_element_type=jnp.float32)
  o += lax.dot_general(h1, w2_ref[128:, :], (((1,), (0,)), ((), ())),
                       precision=_HI, preferred_element_type=jnp.float32)
  o_ref[...] = o * dinv


def _tc_layer2(a0, a1, dega, degb, b1, w2):
  return pl.pallas_call(
      _layer2_body,
      grid=(N // _BR,),
      in_specs=[
          pl.BlockSpec((_BR, 128), lambda i: (i, 0)),
          pl.BlockSpec((_BR, 128), lambda i: (i, 0)),
          pl.BlockSpec((_BR, 16), lambda i: (i, 0)),
          pl.BlockSpec((_BR, 16), lambda i: (i, 0)),
          pl.BlockSpec((1, H), lambda i: (0, 0)),
          pl.BlockSpec((H, C), lambda i: (0, 0)),
      ],
      out_specs=pl.BlockSpec((_BR, C), lambda i: (i, 0)),
      out_shape=jax.ShapeDtypeStruct((N, C), jnp.float32),
  )(a0, a1, dega, degb, b1, w2)


def _finalize_body(aa_ref, ab_ref, hs_ref, dega_ref, degb_ref, b2_ref,
                   logp_ref, hi_ref, lo_ref):
  dinv = lax.rsqrt(dega_ref[:, 0:1] + degb_ref[:, 0:1] + 1.0)
  o = (aa_ref[...] + ab_ref[...] - hs_ref[...]) * dinv + b2_ref[0:1, :]
  m = jnp.max(o, axis=1, keepdims=True)
  sh = o - m
  lse = jnp.log(jnp.sum(jnp.exp(sh), axis=1, keepdims=True))
  logp_ref[...] = sh - lse
  sq = jnp.sum(o * o, axis=1, keepdims=True)
  on = o * lax.rsqrt(sq)
  hi = on.astype(jnp.bfloat16)
  hi_ref[...] = hi
  lo_ref[...] = (on - hi.astype(jnp.float32)).astype(jnp.bfloat16)


def _tc_finalize(aa, ab, hs2, dega, degb, b2):
  return pl.pallas_call(
      _finalize_body,
      grid=(N // _BR,),
      in_specs=[
          pl.BlockSpec((_BR, C), lambda i: (i, 0)),
          pl.BlockSpec((_BR, C), lambda i: (i, 0)),
          pl.BlockSpec((_BR, C), lambda i: (i, 0)),
          pl.BlockSpec((_BR, 16), lambda i: (i, 0)),
          pl.BlockSpec((_BR, 16), lambda i: (i, 0)),
          pl.BlockSpec((1, C), lambda i: (0, 0)),
      ],
      out_specs=[
          pl.BlockSpec((_BR, C), lambda i: (i, 0)),
          pl.BlockSpec((_BR, C), lambda i: (i, 0)),
          pl.BlockSpec((_BR, C), lambda i: (i, 0)),
      ],
      out_shape=[jax.ShapeDtypeStruct((N, C), jnp.float32),
                 jax.ShapeDtypeStruct((N, C), jnp.bfloat16),
                 jax.ShapeDtypeStruct((N, C), jnp.bfloat16)],
  )(aa, ab, hs2, dega, degb, b2)


_BS = 200


def _sim_body(hi_ref, lo_ref, rhs_ref, mask_ref, o_ref):
  dn = (((1,), (0,)), ((), ()))
  hi, lo = hi_ref[...], lo_ref[...]
  lhs = jnp.concatenate([hi, hi, lo, lo], axis=1)
  s = lax.dot_general(lhs, rhs_ref[...], dn, preferred_element_type=jnp.float32)
  o_ref[...] = (1.0 - mask_ref[...]) * s


def _tc_sim(on_hi, on_lo, mask):
  hit = on_hi.T
  lot = on_lo.T
  rhs = jnp.concatenate([hit, lot, hit, lot], axis=0)
  return pl.pallas_call(
      _sim_body,
      grid=(N // _BS,),
      in_specs=[
          pl.BlockSpec((_BS, C), lambda i: (i, 0)),
          pl.BlockSpec((_BS, C), lambda i: (i, 0)),
          pl.BlockSpec((4 * C, N), lambda i: (0, 0)),
          pl.BlockSpec((_BS, N), lambda i: (i, 0)),
      ],
      out_specs=pl.BlockSpec((_BS, N), lambda i: (i, 0)),
      out_shape=jax.ShapeDtypeStruct((N, N), jnp.float32),
  )(on_hi, on_lo, rhs, mask)


def kernel(x, edge_index, mask, W1, b1, W2, b2):
  src = edge_index[0].astype(jnp.int32)
  dst = edge_index[1].astype(jnp.int32)

  extra = EPAD - E
  pad_ids = jnp.arange(extra, dtype=jnp.int32)
  src_p = jnp.concatenate([src, pad_ids % N]).reshape(EROWS, 128)
  dst_p = jnp.concatenate([dst, N + pad_ids % (NRA - N)]).reshape(EROWS, 128)

  ones2d = jnp.ones((128, 16), jnp.float32)
  zeros2d = jnp.zeros((NRD, 16), jnp.float32)
  dega, degb = _sc_degree(dst_p, ones2d, zeros2d)

  hs1_0, hs1_1 = _tc_mm1scale(x, W1, dega, degb)
  a1_0, a1_1 = _make_sc_agg(128)(hs1_0, hs1_1, src_p, dst_p)
  hs2 = _tc_layer2(a1_0, a1_1, dega, degb, b1.reshape(1, H), W2)
  a2_a, a2_b = _make_sc_agg_edges()(hs2, src_p, dst_p)
  logp, on_hi, on_lo = _tc_finalize(a2_a, a2_b, hs2, dega, degb,
                                    b2.reshape(1, C))
  x_dis = _tc_sim(on_hi, on_lo, mask)
  return (logp, x_dis)


# speedup vs baseline: 10.7492x; 1.0169x over previous
import functools

import jax
import jax.numpy as jnp
from jax import lax
from jax.experimental import pallas as pl
from jax.experimental.pallas import tpu as pltpu
from jax.experimental.pallas import tpu_sc as plsc

N = 10000
E = 160000
D = 256
H = 256
C = 64

NSUB = 16
EPAD = 163840
EROWS = EPAD // 128
ROWS_PER_SUB = EROWS // NSUB
CHUNK_ROWS = 8
NCHUNK = ROWS_PER_SUB // CHUNK_ROWS
NRD = 10112
NRA = 10016
IDXROWS = ROWS_PER_SUB // 2
WB = 624
WBTAIL = N - WB * NSUB
DEGWB = NRD // NSUB

_HI = jax.lax.Precision.HIGHEST

@functools.lru_cache(maxsize=None)
def _vector_mesh():
  return plsc.VectorSubcoreMesh(
      core_axis_name="c", subcore_axis_name="s", num_cores=2, num_subcores=NSUB)


def _sc_degree(dst2d, ones2d, zeros2d):
  @functools.partial(
      pl.kernel,
      out_type=(jax.ShapeDtypeStruct((NRD, 16), jnp.float32),
                jax.ShapeDtypeStruct((NRD, 16), jnp.float32)),
      mesh=_vector_mesh(),
      compiler_params=pltpu.CompilerParams(use_tc_tiling_on_sc=False),
      scratch_types=[
          pltpu.VMEM((IDXROWS, 128), jnp.int32),
          pltpu.VMEM((128, 16), jnp.float32),
          pltpu.VMEM_SHARED((NRD, 16), jnp.float32),
      ],
  )
  def deg_kernel(dst_hbm, ones_hbm, zeros_hbm, outa_hbm, outb_hbm,
                 dst_v, ones_v, acc_sh):
    c = lax.axis_index("c")
    s = lax.axis_index("s")

    pltpu.sync_copy(ones_hbm, ones_v)
    pltpu.sync_copy(zeros_hbm.at[pl.ds(s * DEGWB, DEGWB)],
                    acc_sh.at[pl.ds(s * DEGWB, DEGWB)])
    plsc.subcore_barrier()

    base = c * (EROWS // 2) + s * IDXROWS
    pltpu.sync_copy(dst_hbm.at[pl.ds(base, IDXROWS)], dst_v)

    @pl.loop(0, IDXROWS)
    def _(j):
      pltpu.sync_copy(ones_v, acc_sh.at[dst_v.at[j]], add=True)

    plsc.subcore_barrier()

    def wb(out_hbm):
      pltpu.sync_copy(acc_sh.at[pl.ds(s * DEGWB, DEGWB)],
                      out_hbm.at[pl.ds(s * DEGWB, DEGWB)])

    @pl.when(c == 0)
    def _():
      wb(outa_hbm)

    @pl.when(c == 1)
    def _():
      wb(outb_hbm)

  return deg_kernel(dst2d, ones2d, zeros2d)


def _make_sc_agg(fc):
  cp = (None if fc % 128 == 0
        else pltpu.CompilerParams(use_tc_tiling_on_sc=False))

  @functools.partial(
      pl.kernel,
      out_type=(jax.ShapeDtypeStruct((N, fc), jnp.float32),
                jax.ShapeDtypeStruct((N, fc), jnp.float32)),
      mesh=_vector_mesh(),
      compiler_params=cp,
      scratch_types=[
          pltpu.VMEM((IDXROWS, 128), jnp.int32),
          pltpu.VMEM((IDXROWS, 128), jnp.int32),
          pltpu.VMEM((128, fc), jnp.float32),
          pltpu.VMEM((128, fc), jnp.float32),
          pltpu.VMEM_SHARED((NRA, fc), jnp.float32),
          pltpu.SemaphoreType.DMA,
          pltpu.SemaphoreType.DMA,
      ],
  )
  def agg_kernel(h0_hbm, h1_hbm, src_hbm, dst_hbm, out0_hbm, out1_hbm,
                 src_v, dst_v, rows_a, rows_b, acc_sh, sem_a, sem_b):
    c = lax.axis_index("c")
    s = lax.axis_index("s")

    def run(h_hbm, out_hbm):
      pltpu.sync_copy(h_hbm.at[pl.ds(s * WB, WB)], acc_sh.at[pl.ds(s * WB, WB)])

      @pl.when(s == 0)
      def _():
        pltpu.sync_copy(h_hbm.at[pl.ds(WB * NSUB, WBTAIL)],
                        acc_sh.at[pl.ds(WB * NSUB, WBTAIL)])

      plsc.subcore_barrier()

      def start(r, buf, sem):
        pltpu.async_copy(h_hbm.at[src_v.at[r]], buf, sem)

      def wait(buf, sem):
        pltpu.make_async_copy(h_hbm.at[pl.ds(0, 128)], buf, sem).wait()

      def scat(r, buf):
        pltpu.sync_copy(buf, acc_sh.at[dst_v.at[r]], add=True)

      @pl.loop(0, 2)
      def _(p):
        base = s * ROWS_PER_SUB + p * IDXROWS
        pltpu.sync_copy(src_hbm.at[pl.ds(base, IDXROWS)], src_v)
        pltpu.sync_copy(dst_hbm.at[pl.ds(base, IDXROWS)], dst_v)
        start(0, rows_a, sem_a)

        @pl.loop(0, IDXROWS, step=2)
        def _(r):
          start(r + 1, rows_b, sem_b)
          wait(rows_a, sem_a)
          scat(r, rows_a)

          @pl.when(r + 2 < IDXROWS)
          def _():
            start(r + 2, rows_a, sem_a)

          wait(rows_b, sem_b)
          scat(r + 1, rows_b)

      plsc.subcore_barrier()
      pltpu.sync_copy(acc_sh.at[pl.ds(s * WB, WB)], out_hbm.at[pl.ds(s * WB, WB)])

      @pl.when(s == 0)
      def _():
        pltpu.sync_copy(acc_sh.at[pl.ds(WB * NSUB, WBTAIL)],
                        out_hbm.at[pl.ds(WB * NSUB, WBTAIL)])

    @pl.when(c == 0)
    def _():
      run(h0_hbm, out0_hbm)

    @pl.when(c == 1)
    def _():
      run(h1_hbm, out1_hbm)

  return agg_kernel


_make_sc_agg = functools.lru_cache(maxsize=None)(_make_sc_agg)


@functools.lru_cache(maxsize=None)
def _make_sc_agg_edges():
  @functools.partial(
      pl.kernel,
      out_type=(jax.ShapeDtypeStruct((N, C), jnp.float32),
                jax.ShapeDtypeStruct((N, C), jnp.float32)),
      mesh=_vector_mesh(),
      compiler_params=pltpu.CompilerParams(use_tc_tiling_on_sc=False),
      scratch_types=[
          pltpu.VMEM((IDXROWS, 128), jnp.int32),
          pltpu.VMEM((IDXROWS, 128), jnp.int32),
          pltpu.VMEM((128, C), jnp.float32),
          pltpu.VMEM((128, C), jnp.float32),
          pltpu.VMEM_SHARED((NRA, C), jnp.float32),
          pltpu.SemaphoreType.DMA,
          pltpu.SemaphoreType.DMA,
      ],
  )
  def agg_kernel(h_hbm, src_hbm, dst_hbm, oa_hbm, ob_hbm,
                 src_v, dst_v, rows_a, rows_b, acc_sh, sem_a, sem_b):
    c = lax.axis_index("c")
    s = lax.axis_index("s")

    pltpu.sync_copy(h_hbm.at[pl.ds(s * WB, WB)], acc_sh.at[pl.ds(s * WB, WB)])

    @pl.when(s == 0)
    def _():
      pltpu.sync_copy(h_hbm.at[pl.ds(WB * NSUB, WBTAIL)],
                      acc_sh.at[pl.ds(WB * NSUB, WBTAIL)])

    plsc.subcore_barrier()

    def start(r, buf, sem):
      pltpu.async_copy(h_hbm.at[src_v.at[r]], buf, sem)

    def wait(buf, sem):
      pltpu.make_async_copy(h_hbm.at[pl.ds(0, 128)], buf, sem).wait()

    def scat(r, buf):
      pltpu.sync_copy(buf, acc_sh.at[dst_v.at[r]], add=True)

    base = c * (EROWS // 2) + s * IDXROWS
    pltpu.sync_copy(src_hbm.at[pl.ds(base, IDXROWS)], src_v)
    pltpu.sync_copy(dst_hbm.at[pl.ds(base, IDXROWS)], dst_v)
    start(0, rows_a, sem_a)

    @pl.loop(0, IDXROWS, step=2)
    def _(r):
      start(r + 1, rows_b, sem_b)
      wait(rows_a, sem_a)
      scat(r, rows_a)

      @pl.when(r + 2 < IDXROWS)
      def _():
        start(r + 2, rows_a, sem_a)

      wait(rows_b, sem_b)
      scat(r + 1, rows_b)

    plsc.subcore_barrier()

    def wb(out_hbm):
      pltpu.sync_copy(acc_sh.at[pl.ds(s * WB, WB)],
                      out_hbm.at[pl.ds(s * WB, WB)])

      @pl.when(s == 0)
      def _():
        pltpu.sync_copy(acc_sh.at[pl.ds(WB * NSUB, WBTAIL)],
                        out_hbm.at[pl.ds(WB * NSUB, WBTAIL)])

    @pl.when(c == 0)
    def _():
      wb(oa_hbm)

    @pl.when(c == 1)
    def _():
      wb(ob_hbm)

  return agg_kernel


_BR = 1000


def _mm1_body(x_ref, w_ref, dega_ref, degb_ref, o0_ref, o1_ref):
  dinv = lax.rsqrt(dega_ref[:, 0:1] + degb_ref[:, 0:1] + 1.0)
  o = lax.dot_general(x_ref[...], w_ref[...], (((1,), (0,)), ((), ())),
                      precision=_HI, preferred_element_type=jnp.float32)
  o = o * dinv
  o0_ref[...] = o[:, :128]
  o1_ref[...] = o[:, 128:]


def _tc_mm1scale(x, w1, dega, degb):
  return pl.pallas_call(
      _mm1_body,
      grid=(N // _BR,),
      in_specs=[
          pl.BlockSpec((_BR, D), lambda i: (i, 0)),
          pl.BlockSpec((D, H), lambda i: (0, 0)),
          pl.BlockSpec((_BR, 16), lambda i: (i, 0)),
          pl.BlockSpec((_BR, 16), lambda i: (i, 0)),
      ],
      out_specs=[
          pl.BlockSpec((_BR, 128), lambda i: (i, 0)),
          pl.BlockSpec((_BR, 128), lambda i: (i, 0)),
      ],
      out_shape=[jax.ShapeDtypeStruct((N, 128), jnp.float32),
                 jax.ShapeDtypeStruct((N, 128), jnp.float32)],
  )(x, w1, dega, degb)


def _layer2_body(a0_ref, a1_ref, dega_ref, degb_ref, b1_ref, w2_ref, o_ref):
  dinv = lax.rsqrt(dega_ref[:, 0:1] + degb_ref[:, 0:1] + 1.0)
  h0 = jax.nn.relu(a0_ref[...] * dinv + b1_ref[0:1, :128])
  h1 = jax.nn.relu(a1_ref[...] * dinv + b1_ref[0:1, 128:])
  o = lax.dot_general(h0, w2_ref[:128, :], (((1,), (0,)), ((), ())),
                      precision=_HI, preferred_element_type=jnp.float32)
  o += lax.dot_general(h1, w2_ref[128:, :], (((1,), (0,)), ((), ())),
                       precision=_HI, preferred_element_type=jnp.float32)
  o_ref[...] = o * dinv


def _tc_layer2(a0, a1, dega, degb, b1, w2):
  return pl.pallas_call(
      _layer2_body,
      grid=(N // _BR,),
      in_specs=[
          pl.BlockSpec((_BR, 128), lambda i: (i, 0)),
          pl.BlockSpec((_BR, 128), lambda i: (i, 0)),
          pl.BlockSpec((_BR, 16), lambda i: (i, 0)),
          pl.BlockSpec((_BR, 16), lambda i: (i, 0)),
          pl.BlockSpec((1, H), lambda i: (0, 0)),
          pl.BlockSpec((H, C), lambda i: (0, 0)),
      ],
      out_specs=pl.BlockSpec((_BR, C), lambda i: (i, 0)),
      out_shape=jax.ShapeDtypeStruct((N, C), jnp.float32),
  )(a0, a1, dega, degb, b1, w2)


def _finalize_body(aa_ref, ab_ref, hs_ref, dega_ref, degb_ref, b2_ref,
                   logp_ref, hi_ref, lo_ref, rhs_ref):
  dinv = lax.rsqrt(dega_ref[0:N, 0:1] + degb_ref[0:N, 0:1] + 1.0)
  o = (aa_ref[...] + ab_ref[...] - hs_ref[...]) * dinv + b2_ref[0:1, :]
  m = jnp.max(o, axis=1, keepdims=True)
  sh = o - m
  lse = jnp.log(jnp.sum(jnp.exp(sh), axis=1, keepdims=True))
  logp_ref[...] = sh - lse
  sq = jnp.sum(o * o, axis=1, keepdims=True)
  on = o * lax.rsqrt(sq)
  hi = on.astype(jnp.bfloat16)
  hi_ref[...] = hi
  lo_ref[...] = (on - hi.astype(jnp.float32)).astype(jnp.bfloat16)
  ot = on.T
  hit = ot.astype(jnp.bfloat16)
  lot = (ot - hit.astype(jnp.float32)).astype(jnp.bfloat16)
  rhs_ref[0:C, :] = hit
  rhs_ref[C:2 * C, :] = lot
  rhs_ref[2 * C:3 * C, :] = hit
  rhs_ref[3 * C:, :] = lot


def _tc_finalize(aa, ab, hs2, dega, degb, b2):
  return pl.pallas_call(
      _finalize_body,
      out_shape=[jax.ShapeDtypeStruct((N, C), jnp.float32),
                 jax.ShapeDtypeStruct((N, C), jnp.bfloat16),
                 jax.ShapeDtypeStruct((N, C), jnp.bfloat16),
                 jax.ShapeDtypeStruct((4 * C, N), jnp.bfloat16)],
  )(aa, ab, hs2, dega, degb, b2)


_BS = 200


def _sim_body(hi_ref, lo_ref, rhs_ref, mask_ref, o_ref):
  dn = (((1,), (0,)), ((), ()))
  hi, lo = hi_ref[...], lo_ref[...]
  lhs = jnp.concatenate([hi, hi, lo, lo], axis=1)
  s = lax.dot_general(lhs, rhs_ref[...], dn, preferred_element_type=jnp.float32)
  o_ref[...] = (1.0 - mask_ref[...]) * s


def _tc_sim(on_hi, on_lo, rhs, mask):
  return pl.pallas_call(
      _sim_body,
      grid=(N // _BS,),
      in_specs=[
          pl.BlockSpec((_BS, C), lambda i: (i, 0)),
          pl.BlockSpec((_BS, C), lambda i: (i, 0)),
          pl.BlockSpec((4 * C, N), lambda i: (0, 0)),
          pl.BlockSpec((_BS, N), lambda i: (i, 0)),
      ],
      out_specs=pl.BlockSpec((_BS, N), lambda i: (i, 0)),
      out_shape=jax.ShapeDtypeStruct((N, N), jnp.float32),
  )(on_hi, on_lo, rhs, mask)


def kernel(x, edge_index, mask, W1, b1, W2, b2):
  src = edge_index[0].astype(jnp.int32)
  dst = edge_index[1].astype(jnp.int32)

  extra = EPAD - E
  pad_ids = jnp.arange(extra, dtype=jnp.int32)
  src_p = jnp.concatenate([src, pad_ids]).reshape(EROWS, 128)
  dst_p = jnp.concatenate([dst, N + (pad_ids & 15)]).reshape(EROWS, 128)

  ones2d = jnp.ones((128, 16), jnp.float32)
  zeros2d = jnp.zeros((NRD, 16), jnp.float32)
  dega, degb = _sc_degree(dst_p, ones2d, zeros2d)

  hs1_0, hs1_1 = _tc_mm1scale(x, W1, dega, degb)
  a1_0, a1_1 = _make_sc_agg(128)(hs1_0, hs1_1, src_p, dst_p)
  hs2 = _tc_layer2(a1_0, a1_1, dega, degb, b1.reshape(1, H), W2)
  a2_a, a2_b = _make_sc_agg_edges()(hs2, src_p, dst_p)
  logp, on_hi, on_lo, rhs = _tc_finalize(a2_a, a2_b, hs2, dega, degb,
                                         b2.reshape(1, C))
  x_dis = _tc_sim(on_hi, on_lo, rhs, mask)
  return (logp, x_dis)
